# baseline (speedup 1.0000x reference)
import functools

import jax
import jax.numpy as jnp
from jax import lax
from jax.experimental import pallas as pl
from jax.experimental.pallas import tpu as pltpu
from jax.experimental.pallas import tpu_sc as plsc

N = 10000
E = 320000
D = 128
NEG_SLOPE = 0.2

NC = 2
NS = 16
L = 16
NW = NC * NS
C = 80
EWP = 10080
EP = NW * EWP
NCHUNK = EWP // C
NPAIR = NCHUNK // 2
RPT8 = 624
DEN_R = 79


def _ab_body(h_ref, w_ref, o_ref):
    o_ref[...] = jnp.dot(h_ref[...], w_ref[...],
                         preferred_element_type=jnp.float32)


def _ab_call(h, w_pad):
    blk = 1000
    return pl.pallas_call(
        _ab_body,
        grid=(N // blk,),
        in_specs=[
            pl.BlockSpec((blk, D), lambda i: (i, 0)),
            pl.BlockSpec((D, 128), lambda i: (0, 0)),
        ],
        out_specs=pl.BlockSpec((blk, 128), lambda i: (i, 0)),
        out_shape=jax.ShapeDtypeStruct((N, 128), jnp.float32),
    )(h, w_pad)


def _sc_body(h_hbm, ei_hbm, a_hbm, b_hbm, part_hbm, den_hbm,
             src0, dst0, src1, dst1, a_v, b_v, den_v, p_v, rows0, rows1, acc,
             gsem0, gsem1, isem0, isem1):
    cid = lax.axis_index("c")
    sid = lax.axis_index("s")
    wid = sid * NC + cid
    ebase = wid * EWP

    pltpu.sync_copy(a_hbm, a_v)
    pltpu.sync_copy(b_hbm, b_v)

    def _zden(i, carry):
        for j in range(128 // L):
            den_v[i, pl.ds(j * L, L)] = jnp.zeros((L,), jnp.float32)
        return carry
    lax.fori_loop(0, DEN_R, _zden, 0)

    def _zrow(i, carry):
        for j in range(D // L):
            rows0[i, pl.ds(j * L, L)] = jnp.zeros((L,), jnp.float32)
        return carry
    lax.fori_loop(0, C, _zrow, 0)

    base = sid * RPT8
    zcps = [pltpu.async_copy(rows0, acc.at[pl.ds(base + kk * C, C)], gsem0)
            for kk in range(RPT8 // C)]
    zcps.append(pltpu.async_copy(rows0.at[pl.ds(0, RPT8 % C)],
                                 acc.at[pl.ds(base + (RPT8 // C) * C,
                                              RPT8 % C)], gsem0))

    @pl.when(sid == NS - 1)
    def _tail_zero():
        pltpu.sync_copy(rows0.at[pl.ds(0, N - NS * RPT8)],
                        acc.at[pl.ds(NS * RPT8, N - NS * RPT8)])
    for cp in zcps:
        cp.wait()
    plsc.subcore_barrier()

    def _process(ci, src_r, dst_r, rows_r, cp):
        for g in range(C // L):
            sv = src_r[pl.ds(g * L, L)]
            dv = dst_r[pl.ds(g * L, L)]
            e = plsc.load_gather(a_v, [sv]) + plsc.load_gather(b_v, [dv])
            e = jnp.where(e >= 0, e, NEG_SLOPE * e)
            p = jnp.exp(e)
            gid = ebase + ci * C + g * L + lax.iota(jnp.int32, L)
            p = jnp.where(gid < E, p, 0.0)
            p_v[pl.ds(g * L, L)] = p
            plsc.addupdate_scatter(
                den_v, [lax.shift_right_logical(dv, 7),
                        jnp.bitwise_and(dv, 127)], p)
        cp.wait()

        def _scale(i, carry2):
            pi = p_v[pl.ds(i, L)][0]
            for j in range(D // L):
                rows_r[i, pl.ds(j * L, L)] = rows_r[i, pl.ds(j * L, L)] * pi
            return carry2
        lax.fori_loop(0, C, _scale, 0)
        pltpu.sync_copy(rows_r, acc.at[dst_r], add=True)

    pltpu.async_copy(ei_hbm.at[0, pl.ds(ebase, C)], src0, isem0).wait()
    pltpu.async_copy(ei_hbm.at[1, pl.ds(ebase, C)], dst0, isem0).wait()

    def _pair(k, carry):
        c0 = 2 * k
        cps = [pltpu.async_copy(ei_hbm.at[0, pl.ds(ebase + (c0 + 1) * C, C)],
                                src1, isem1),
               pltpu.async_copy(ei_hbm.at[1, pl.ds(ebase + (c0 + 1) * C, C)],
                                dst1, isem1)]
        cp = pltpu.async_copy(h_hbm.at[src0], rows0, gsem0)
        _process(c0, src0, dst0, rows0, cp)
        for c in cps:
            c.wait()

        @pl.when(k < NPAIR - 1)
        def _pf0():
            pltpu.async_copy(ei_hbm.at[0, pl.ds(ebase + (c0 + 2) * C, C)],
                             src0, isem0)
            pltpu.async_copy(ei_hbm.at[1, pl.ds(ebase + (c0 + 2) * C, C)],
                             dst0, isem0)
        cp1 = pltpu.async_copy(h_hbm.at[src1], rows1, gsem1)
        _process(c0 + 1, src1, dst1, rows1, cp1)

        @pl.when(k < NPAIR - 1)
        def _w0():
            pltpu.make_async_copy(ei_hbm.at[0, pl.ds(0, C)], src0,
                                  isem0).wait()
            pltpu.make_async_copy(ei_hbm.at[1, pl.ds(0, C)], dst0,
                                  isem0).wait()
        return carry
    lax.fori_loop(0, NPAIR, _pair, 0)

    plsc.subcore_barrier()

    pltpu.sync_copy(acc.at[pl.ds(base, RPT8)],
                    part_hbm.at[cid, pl.ds(base, RPT8)])

    @pl.when(sid == NS - 1)
    def _tail_out():
        pltpu.sync_copy(acc.at[pl.ds(NS * RPT8, N - NS * RPT8)],
                        part_hbm.at[cid, pl.ds(NS * RPT8, N - NS * RPT8)])

    pltpu.sync_copy(den_v, den_hbm.at[wid])


def _sc_call(h, ei_p, a, b):
    mesh = plsc.VectorSubcoreMesh(core_axis_name="c", subcore_axis_name="s",
                                  num_cores=NC, num_subcores=NS)
    fn = pl.kernel(
        _sc_body,
        out_type=(
            jax.ShapeDtypeStruct((NC, N, D), jnp.float32),
            jax.ShapeDtypeStruct((NW, DEN_R, 128), jnp.float32),
        ),
        mesh=mesh,
        compiler_params=pltpu.CompilerParams(needs_layout_passes=False,
                                             use_tc_tiling_on_sc=False),
        scratch_types=(
            pltpu.VMEM((C,), jnp.int32),
            pltpu.VMEM((C,), jnp.int32),
            pltpu.VMEM((C,), jnp.int32),
            pltpu.VMEM((C,), jnp.int32),
            pltpu.VMEM((N,), jnp.float32),
            pltpu.VMEM((N,), jnp.float32),
            pltpu.VMEM((DEN_R, 128), jnp.float32),
            pltpu.VMEM((C + L,), jnp.float32),
            pltpu.VMEM((C, D), jnp.float32),
            pltpu.VMEM((C, D), jnp.float32),
            pltpu.VMEM_SHARED((N, D), jnp.float32),
            pltpu.SemaphoreType.DMA,
            pltpu.SemaphoreType.DMA,
            pltpu.SemaphoreType.DMA,
            pltpu.SemaphoreType.DMA,
        ),
    )
    return fn(h, ei_p, a, b)


def _fin_body(p0_ref, p1_ref, d_ref, o_ref):
    s = p0_ref[0] + p1_ref[0]
    den = jnp.sum(d_ref[...], axis=1)
    o_ref[...] = s / jnp.maximum(den, 1e-9)[:, None]


def _fin_call(partials, denoms_t):
    blk = 400
    return pl.pallas_call(
        _fin_body,
        grid=(N // blk,),
        in_specs=[
            pl.BlockSpec((1, blk, D), lambda i: (0, i, 0)),
            pl.BlockSpec((1, blk, D), lambda i: (1, i, 0)),
            pl.BlockSpec((blk, NW), lambda i: (i, 0)),
        ],
        out_specs=pl.BlockSpec((blk, D), lambda i: (i, 0)),
        out_shape=jax.ShapeDtypeStruct((N, D), jnp.float32),
    )(partials, partials, denoms_t)


@jax.jit
def kernel(h, edge_index, W_att):
    w_row = W_att[0]
    w_pad = jnp.zeros((D, 128), jnp.float32)
    w_pad = w_pad.at[:, 0].set(w_row[:D]).at[:, 1].set(w_row[D:])
    ab = _ab_call(h, w_pad)
    a = ab[:, 0]
    b = ab[:, 1]
    ei_p = jnp.pad(edge_index, ((0, 0), (0, EP - E)))
    partials, denoms = _sc_call(h, ei_p, a, b)
    den_t = denoms.reshape(NW, DEN_R * 128)[:, :N].T
    return _fin_call(partials, den_t)


# speedup vs baseline: 1.2849x; 1.2849x over previous
import functools

import jax
import jax.numpy as jnp
from jax import lax
from jax.experimental import pallas as pl
from jax.experimental.pallas import tpu as pltpu
from jax.experimental.pallas import tpu_sc as plsc

N = 10000
E = 320000
D = 128
NEG_SLOPE = 0.2

NC = 2
NS = 16
L = 16
NW = NC * NS
EW = E // NW
C = 80
NCHUNK = EW // C
RPT8 = 624
DEN_R = 80
TBL = DEN_R * 128


def _ab_body(h_ref, w_ref, o_ref):
    o_ref[...] = jnp.dot(h_ref[...], w_ref[...],
                         preferred_element_type=jnp.float32)


def _ab_call(h, w_pad):
    blk = 1000
    return pl.pallas_call(
        _ab_body,
        grid=(N // blk,),
        in_specs=[
            pl.BlockSpec((blk, D), lambda i: (i, 0)),
            pl.BlockSpec((D, 128), lambda i: (0, 0)),
        ],
        out_specs=pl.BlockSpec((blk, 128), lambda i: (i, 0)),
        out_shape=jax.ShapeDtypeStruct((N, 128), jnp.float32),
    )(h, w_pad)


def _sc_body(h_hbm, src_hbm, dst_hbm, a_hbm, b_hbm, part_hbm, den_hbm,
             src_v, dst_v, a_v, b_v, den_v, p_v, rows_v, acc, sem, isem):
    cid = lax.axis_index("c")
    sid = lax.axis_index("s")
    wid = sid * NC + cid
    ebase = wid * EW

    pltpu.sync_copy(a_hbm, a_v.at[pl.ds(0, N)])
    pltpu.sync_copy(b_hbm, b_v.at[pl.ds(0, N)])

    def _zden(i, carry):
        den_v[pl.ds(i * L, L)] = jnp.zeros((L,), jnp.float32)
        return carry
    lax.fori_loop(0, TBL // L, _zden, 0)

    def _zrow(i, carry):
        for j in range(D // L):
            rows_v[i, pl.ds(j * L, L)] = jnp.zeros((L,), jnp.float32)
        return carry
    lax.fori_loop(0, C, _zrow, 0)
    base = sid * RPT8
    for k in range(RPT8 // C):
        pltpu.sync_copy(rows_v, acc.at[pl.ds(base + k * C, C)])
    rem = RPT8 % C
    if rem:
        pltpu.sync_copy(rows_v.at[pl.ds(0, rem)],
                        acc.at[pl.ds(base + (RPT8 // C) * C, rem)])

    @pl.when(sid == NS - 1)
    def _tail_zero():
        pltpu.sync_copy(rows_v.at[pl.ds(0, N - NS * RPT8)],
                        acc.at[pl.ds(NS * RPT8, N - NS * RPT8)])
    plsc.subcore_barrier()

    pltpu.async_copy(src_hbm.at[pl.ds(ebase, C)], src_v.at[0], isem).wait()
    pltpu.async_copy(dst_hbm.at[pl.ds(ebase, C)], dst_v.at[0], isem).wait()

    def _chunk(ci, carry):
        slot = lax.rem(ci, 2)
        nslot = 1 - slot
        nci = jnp.minimum(ci + 1, NCHUNK - 1)
        cpn_s = pltpu.async_copy(src_hbm.at[pl.ds(ebase + nci * C, C)],
                                 src_v.at[nslot], isem)
        cpn_d = pltpu.async_copy(dst_hbm.at[pl.ds(ebase + nci * C, C)],
                                 dst_v.at[nslot], isem)
        cp = pltpu.async_copy(h_hbm.at[src_v.at[slot]], rows_v, sem)
        for g in range(C // L):
            sv = src_v[slot, pl.ds(g * L, L)]
            dv = dst_v[slot, pl.ds(g * L, L)]
            e = plsc.load_gather(a_v, [sv]) + plsc.load_gather(b_v, [dv])
            e = jnp.where(e >= 0, e, NEG_SLOPE * e)
            p = jnp.exp(e)
            p_v[pl.ds(g * L, L)] = p
            plsc.addupdate_scatter(den_v, [dv], p)
        cp.wait()

        def _scale(i, carry2):
            pi = p_v[pl.ds(i, L)][0]
            for j in range(D // L):
                rows_v[i, pl.ds(j * L, L)] = rows_v[i, pl.ds(j * L, L)] * pi
            return carry2
        lax.fori_loop(0, C, _scale, 0)

        pltpu.sync_copy(rows_v, acc.at[dst_v.at[slot]], add=True)
        cpn_s.wait()
        cpn_d.wait()
        return carry
    lax.fori_loop(0, NCHUNK, _chunk, 0)

    plsc.subcore_barrier()

    pltpu.sync_copy(acc.at[pl.ds(base, RPT8)],
                    part_hbm.at[cid, pl.ds(base, RPT8)])

    @pl.when(sid == NS - 1)
    def _tail_out():
        pltpu.sync_copy(acc.at[pl.ds(NS * RPT8, N - NS * RPT8)],
                        part_hbm.at[cid, pl.ds(NS * RPT8, N - NS * RPT8)])

    pltpu.sync_copy(den_v.at[pl.ds(0, TBL)], den_hbm.at[wid])


def _sc_call(h, src, dst, a, b):
    mesh = plsc.VectorSubcoreMesh(core_axis_name="c", subcore_axis_name="s",
                                  num_cores=NC, num_subcores=NS)
    fn = pl.kernel(
        _sc_body,
        out_type=(
            jax.ShapeDtypeStruct((NC, N, D), jnp.float32),
            jax.ShapeDtypeStruct((NW, TBL), jnp.float32),
        ),
        mesh=mesh,
        compiler_params=pltpu.CompilerParams(needs_layout_passes=False,
                                             use_tc_tiling_on_sc=False),
        scratch_types=(
            pltpu.VMEM((2, C), jnp.int32),
            pltpu.VMEM((2, C), jnp.int32),
            pltpu.VMEM((TBL,), jnp.float32),
            pltpu.VMEM((TBL,), jnp.float32),
            pltpu.VMEM((TBL,), jnp.float32),
            pltpu.VMEM((C + L,), jnp.float32),
            pltpu.VMEM((C, D), jnp.float32),
            pltpu.VMEM_SHARED((N, D), jnp.float32),
            pltpu.SemaphoreType.DMA,
            pltpu.SemaphoreType.DMA,
        ),
    )
    return fn(h, src, dst, a, b)


def _fin_body(p0_ref, p1_ref, d_ref, o_ref):
    s = p0_ref[0] + p1_ref[0]
    den = jnp.sum(d_ref[...], axis=1)
    o_ref[...] = s / jnp.maximum(den, 1e-9)[:, None]


def _fin_call(partials, denoms_t):
    blk = 400
    return pl.pallas_call(
        _fin_body,
        grid=(N // blk,),
        in_specs=[
            pl.BlockSpec((1, blk, D), lambda i: (0, i, 0)),
            pl.BlockSpec((1, blk, D), lambda i: (1, i, 0)),
            pl.BlockSpec((blk, NW), lambda i: (i, 0)),
        ],
        out_specs=pl.BlockSpec((blk, D), lambda i: (i, 0)),
        out_shape=jax.ShapeDtypeStruct((N, D), jnp.float32),
    )(partials, partials, denoms_t)


@jax.jit
def kernel(h, edge_index, W_att):
    w_row = W_att[0]
    w_pad = jnp.zeros((D, 128), jnp.float32)
    w_pad = w_pad.at[:, 0].set(w_row[:D]).at[:, 1].set(w_row[D:])
    ab = _ab_call(h, w_pad)
    a = ab[:, 0]
    b = ab[:, 1]
    partials, denoms = _sc_call(h, edge_index[0], edge_index[1], a, b)
    den_t = denoms[:, :N].T
    return _fin_call(partials, den_t)


# speedup vs baseline: 1.4691x; 1.1433x over previous
import functools

import jax
import jax.numpy as jnp
from jax import lax
from jax.experimental import pallas as pl
from jax.experimental.pallas import tpu as pltpu
from jax.experimental.pallas import tpu_sc as plsc

N = 10000
E = 320000
D = 128
NEG_SLOPE = 0.2

NC = 2
NS = 16
L = 16
NW = NC * NS
EW = E // NW
C = 80
NCHUNK = EW // C
NPAIR = NCHUNK // 2
RPT8 = 624
TBL = 10240
ZB = TBL // NS


def _ab_body(h_ref, w_ref, o_ref):
    o_ref[...] = jnp.dot(h_ref[...], w_ref[...],
                         preferred_element_type=jnp.float32)


def _ab_call(h, w_pad):
    blk = 1000
    return pl.pallas_call(
        _ab_body,
        grid=(N // blk,),
        in_specs=[
            pl.BlockSpec((blk, D), lambda i: (i, 0)),
            pl.BlockSpec((D, 128), lambda i: (0, 0)),
        ],
        out_specs=pl.BlockSpec((blk, 128), lambda i: (i, 0)),
        out_shape=jax.ShapeDtypeStruct((N, 128), jnp.float32),
    )(h, w_pad)


def _sc_body(h_hbm, src_hbm, dst_hbm, a_hbm, b_hbm, part_hbm, den_hbm,
             src0, dst0, src1, dst1, a_v, b_v, p_v, zbuf, rows0, rows1,
             acc, den_sh, gsem0, gsem1, isem0, isem1):
    cid = lax.axis_index("c")
    sid = lax.axis_index("s")
    wid = sid * NC + cid
    ebase = wid * EW

    pltpu.sync_copy(a_hbm, a_v.at[pl.ds(0, N)])
    pltpu.sync_copy(b_hbm, b_v.at[pl.ds(0, N)])

    def _zrow(i, carry):
        for j in range(D // L):
            rows0[i, pl.ds(j * L, L)] = jnp.zeros((L,), jnp.float32)
        return carry
    lax.fori_loop(0, C, _zrow, 0)

    def _zzb(i, carry):
        zbuf[pl.ds(i * L, L)] = jnp.zeros((L,), jnp.float32)
        return carry
    lax.fori_loop(0, ZB // L, _zzb, 0)

    base = sid * RPT8
    zcps = [pltpu.async_copy(rows0, acc.at[pl.ds(base + k * C, C)], gsem0)
            for k in range(RPT8 // C)]
    zcps.append(pltpu.async_copy(rows0.at[pl.ds(0, RPT8 % C)],
                                 acc.at[pl.ds(base + (RPT8 // C) * C,
                                              RPT8 % C)], gsem0))
    zcps.append(pltpu.async_copy(zbuf, den_sh.at[pl.ds(sid * ZB, ZB)], gsem0))

    @pl.when(sid == NS - 1)
    def _tail_zero():
        pltpu.sync_copy(rows0.at[pl.ds(0, N - NS * RPT8)],
                        acc.at[pl.ds(NS * RPT8, N - NS * RPT8)])
    for cp in zcps:
        cp.wait()
    plsc.subcore_barrier()

    def _process(src_r, dst_r, rows_r):
        for g in range(C // L):
            sv = src_r[pl.ds(g * L, L)]
            dv = dst_r[pl.ds(g * L, L)]
            e = plsc.load_gather(a_v, [sv]) + plsc.load_gather(b_v, [dv])
            e = jnp.where(e >= 0, e, NEG_SLOPE * e)
            p_v[pl.ds(g * L, L)] = jnp.exp(e)

        def _scale(i, carry2):
            pi = p_v[pl.ds(i, L)][0]
            for j in range(D // L):
                rows_r[i, pl.ds(j * L, L)] = rows_r[i, pl.ds(j * L, L)] * pi
            return carry2
        lax.fori_loop(0, C, _scale, 0)

        pltpu.sync_copy(p_v.at[pl.ds(0, C)], den_sh.at[dst_r], add=True)
        pltpu.sync_copy(rows_r, acc.at[dst_r], add=True)

    pltpu.async_copy(src_hbm.at[pl.ds(ebase, C)], src0, isem0).wait()
    pltpu.async_copy(dst_hbm.at[pl.ds(ebase, C)], dst0, isem0).wait()
    pltpu.async_copy(src_hbm.at[pl.ds(ebase + C, C)], src1, isem1)
    pltpu.async_copy(dst_hbm.at[pl.ds(ebase + C, C)], dst1, isem1)
    pltpu.async_copy(h_hbm.at[src0], rows0, gsem0)

    def _pair(k, carry):
        c0 = 2 * k
        pltpu.make_async_copy(src_hbm.at[pl.ds(0, C)], src1, isem1).wait()
        pltpu.make_async_copy(dst_hbm.at[pl.ds(0, C)], dst1, isem1).wait()
        pltpu.async_copy(h_hbm.at[src1], rows1, gsem1)
        pltpu.make_async_copy(h_hbm.at[pl.ds(0, C)], rows0, gsem0).wait()
        _process(src0, dst0, rows0)
        pltpu.async_copy(src_hbm.at[pl.ds(ebase + (c0 + 2) * C, C)],
                         src0, isem0)
        pltpu.async_copy(dst_hbm.at[pl.ds(ebase + (c0 + 2) * C, C)],
                         dst0, isem0)
        pltpu.make_async_copy(h_hbm.at[pl.ds(0, C)], rows1, gsem1).wait()
        _process(src1, dst1, rows1)

        @pl.when(k < NPAIR - 1)
        def _pf1():
            pltpu.async_copy(src_hbm.at[pl.ds(ebase + (c0 + 3) * C, C)],
                             src1, isem1)
            pltpu.async_copy(dst_hbm.at[pl.ds(ebase + (c0 + 3) * C, C)],
                             dst1, isem1)
        pltpu.make_async_copy(src_hbm.at[pl.ds(0, C)], src0, isem0).wait()
        pltpu.make_async_copy(dst_hbm.at[pl.ds(0, C)], dst0, isem0).wait()
        pltpu.async_copy(h_hbm.at[src0], rows0, gsem0)
        return carry
    lax.fori_loop(0, NPAIR, _pair, 0)

    pltpu.make_async_copy(h_hbm.at[pl.ds(0, C)], rows0, gsem0).wait()
    _process(src0, dst0, rows0)

    plsc.subcore_barrier()

    pltpu.sync_copy(acc.at[pl.ds(base, RPT8)],
                    part_hbm.at[cid, pl.ds(base, RPT8)])

    @pl.when(sid == NS - 1)
    def _tail_out():
        pltpu.sync_copy(acc.at[pl.ds(NS * RPT8, N - NS * RPT8)],
                        part_hbm.at[cid, pl.ds(NS * RPT8, N - NS * RPT8)])

    pltpu.sync_copy(den_sh.at[pl.ds(sid * ZB, ZB)],
                    den_hbm.at[cid, pl.ds(sid * ZB, ZB)])


def _sc_call(h, src, dst, a, b):
    mesh = plsc.VectorSubcoreMesh(core_axis_name="c", subcore_axis_name="s",
                                  num_cores=NC, num_subcores=NS)
    fn = pl.kernel(
        _sc_body,
        out_type=(
            jax.ShapeDtypeStruct((NC, N, D), jnp.float32),
            jax.ShapeDtypeStruct((NC, TBL), jnp.float32),
        ),
        mesh=mesh,
        compiler_params=pltpu.CompilerParams(needs_layout_passes=False,
                                             use_tc_tiling_on_sc=False),
        scratch_types=(
            pltpu.VMEM((C,), jnp.int32),
            pltpu.VMEM((C,), jnp.int32),
            pltpu.VMEM((C,), jnp.int32),
            pltpu.VMEM((C,), jnp.int32),
            pltpu.VMEM((TBL,), jnp.float32),
            pltpu.VMEM((TBL,), jnp.float32),
            pltpu.VMEM((C + L,), jnp.float32),
            pltpu.VMEM((ZB,), jnp.float32),
            pltpu.VMEM((C, D), jnp.float32),
            pltpu.VMEM((C, D), jnp.float32),
            pltpu.VMEM_SHARED((N, D), jnp.float32),
            pltpu.VMEM_SHARED((TBL,), jnp.float32),
            pltpu.SemaphoreType.DMA,
            pltpu.SemaphoreType.DMA,
            pltpu.SemaphoreType.DMA,
            pltpu.SemaphoreType.DMA,
        ),
    )
    return fn(h, src, dst, a, b)


def _fin_body(p0_ref, p1_ref, d_ref, o_ref):
    s = p0_ref[0] + p1_ref[0]
    den = jnp.sum(d_ref[...], axis=1)
    o_ref[...] = s / jnp.maximum(den, 1e-9)[:, None]


def _fin_call(partials, denoms_t):
    blk = 400
    return pl.pallas_call(
        _fin_body,
        grid=(N // blk,),
        in_specs=[
            pl.BlockSpec((1, blk, D), lambda i: (0, i, 0)),
            pl.BlockSpec((1, blk, D), lambda i: (1, i, 0)),
            pl.BlockSpec((blk, NC), lambda i: (i, 0)),
        ],
        out_specs=pl.BlockSpec((blk, D), lambda i: (i, 0)),
        out_shape=jax.ShapeDtypeStruct((N, D), jnp.float32),
    )(partials, partials, denoms_t)


@jax.jit
def kernel(h, edge_index, W_att):
    w_row = W_att[0]
    w_pad = jnp.zeros((D, 128), jnp.float32)
    w_pad = w_pad.at[:, 0].set(w_row[:D]).at[:, 1].set(w_row[D:])
    ab = _ab_call(h, w_pad)
    a = ab[:, 0]
    b = ab[:, 1]
    partials, denoms = _sc_call(h, edge_index[0], edge_index[1], a, b)
    den_t = denoms.T[:N]
    return _fin_call(partials, den_t)


# speedup vs baseline: 1.8012x; 1.2261x over previous
import functools

import jax
import jax.numpy as jnp
from jax import lax
from jax.experimental import pallas as pl
from jax.experimental.pallas import tpu as pltpu
from jax.experimental.pallas import tpu_sc as plsc

N = 10000
E = 320000
D = 128
NEG_SLOPE = 0.2

NC = 2
NS = 16
L = 16
NW = NC * NS
EW = E // NW
C = 80
NCHUNK = EW // C
NPAIR = NCHUNK // 2
RPT8 = 624
TBL = 10240
ZB = TBL // NS


def _ab_body(h_ref, w_ref, o_ref):
    o_ref[...] = jnp.dot(h_ref[...], w_ref[...],
                         preferred_element_type=jnp.float32)


def _ab_call(h, w_pad):
    blk = 1000
    return pl.pallas_call(
        _ab_body,
        grid=(N // blk,),
        in_specs=[
            pl.BlockSpec((blk, D), lambda i: (i, 0)),
            pl.BlockSpec((D, 128), lambda i: (0, 0)),
        ],
        out_specs=pl.BlockSpec((blk, 128), lambda i: (i, 0)),
        out_shape=jax.ShapeDtypeStruct((N, 128), jnp.float32),
    )(h, w_pad)


def _sc_body(h_hbm, src_hbm, dst_hbm, a_hbm, b_hbm, part_hbm, den_hbm,
             src0, dst0, src1, dst1, a_v, b_v, zbuf, rows0, rows1,
             scidx0, scidx1, pden0, pden1, acc, den_sh,
             gsem0, gsem1, isem0, isem1, ssem0, ssem1, dsem0, dsem1):
    cid = lax.axis_index("c")
    sid = lax.axis_index("s")
    wid = sid * NC + cid
    ebase = wid * EW

    pltpu.sync_copy(a_hbm, a_v.at[pl.ds(0, N)])
    pltpu.sync_copy(b_hbm, b_v.at[pl.ds(0, N)])

    def _zrow(i, carry):
        for j in range(D // L):
            rows0[i, pl.ds(j * L, L)] = jnp.zeros((L,), jnp.float32)
        return carry
    lax.fori_loop(0, C, _zrow, 0)

    def _zzb(i, carry):
        zbuf[pl.ds(i * L, L)] = jnp.zeros((L,), jnp.float32)
        return carry
    lax.fori_loop(0, ZB // L, _zzb, 0)

    base = sid * RPT8
    zcps = [pltpu.async_copy(rows0, acc.at[pl.ds(base + k * C, C)], gsem0)
            for k in range(RPT8 // C)]
    zcps.append(pltpu.async_copy(rows0.at[pl.ds(0, RPT8 % C)],
                                 acc.at[pl.ds(base + (RPT8 // C) * C,
                                              RPT8 % C)], gsem0))
    zcps.append(pltpu.async_copy(zbuf, den_sh.at[pl.ds(sid * ZB, ZB)], gsem0))

    @pl.when(sid == NS - 1)
    def _tail_zero():
        pltpu.sync_copy(rows0.at[pl.ds(0, N - NS * RPT8)],
                        acc.at[pl.ds(NS * RPT8, N - NS * RPT8)])
    for cp in zcps:
        cp.wait()
    plsc.subcore_barrier()

    def _process(src_r, dst_r, rows_r, scidx, pden, ssem, dsem):
        for g in range(C // L):
            sv = src_r[pl.ds(g * L, L)]
            dv = dst_r[pl.ds(g * L, L)]
            e = plsc.load_gather(a_v, [sv]) + plsc.load_gather(b_v, [dv])
            e = jnp.where(e >= 0, e, NEG_SLOPE * e)
            pden[pl.ds(g * L, L)] = jnp.exp(e)
            scidx[pl.ds(g * L, L)] = dv
        pltpu.async_copy(pden.at[pl.ds(0, C)], den_sh.at[scidx], dsem,
                         add=True)

        def _scale(i, carry2):
            pi = pden[pl.ds(i, L)][0]
            for j in range(D // L):
                rows_r[i, pl.ds(j * L, L)] = rows_r[i, pl.ds(j * L, L)] * pi
            return carry2
        lax.fori_loop(0, C, _scale, 0)

        pltpu.async_copy(rows_r, acc.at[scidx], ssem, add=True)

    pltpu.async_copy(src_hbm.at[pl.ds(ebase, C)], src0, isem0).wait()
    pltpu.async_copy(dst_hbm.at[pl.ds(ebase, C)], dst0, isem0).wait()
    pltpu.async_copy(src_hbm.at[pl.ds(ebase + C, C)], src1, isem1)
    pltpu.async_copy(dst_hbm.at[pl.ds(ebase + C, C)], dst1, isem1)
    pltpu.async_copy(h_hbm.at[src0], rows0, gsem0)

    def _drain_rows(rows_r, ssem):
        pltpu.make_async_copy(h_hbm.at[pl.ds(0, C)], rows_r, ssem).wait()

    def _drain_p(pden, dsem):
        pltpu.make_async_copy(a_hbm.at[pl.ds(0, C)], pden.at[pl.ds(0, C)],
                              dsem).wait()

    def _pair(k, carry):
        c0 = 2 * k

        @pl.when(k > 0)
        def _dr1():
            _drain_rows(rows1, ssem1)
            _drain_p(pden1, dsem1)

        pltpu.make_async_copy(src_hbm.at[pl.ds(0, C)], src1, isem1).wait()
        pltpu.make_async_copy(dst_hbm.at[pl.ds(0, C)], dst1, isem1).wait()
        pltpu.async_copy(h_hbm.at[src1], rows1, gsem1)

        @pl.when(k > 0)
        def _dr0():
            _drain_p(pden0, dsem0)

        pltpu.make_async_copy(h_hbm.at[pl.ds(0, C)], rows0, gsem0).wait()
        _process(src0, dst0, rows0, scidx0, pden0, ssem0, dsem0)
        pltpu.async_copy(src_hbm.at[pl.ds(ebase + (c0 + 2) * C, C)],
                         src0, isem0)
        pltpu.async_copy(dst_hbm.at[pl.ds(ebase + (c0 + 2) * C, C)],
                         dst0, isem0)
        pltpu.make_async_copy(h_hbm.at[pl.ds(0, C)], rows1, gsem1).wait()
        _process(src1, dst1, rows1, scidx1, pden1, ssem1, dsem1)

        @pl.when(k < NPAIR - 1)
        def _pf1():
            pltpu.async_copy(src_hbm.at[pl.ds(ebase + (c0 + 3) * C, C)],
                             src1, isem1)
            pltpu.async_copy(dst_hbm.at[pl.ds(ebase + (c0 + 3) * C, C)],
                             dst1, isem1)
        pltpu.make_async_copy(src_hbm.at[pl.ds(0, C)], src0, isem0).wait()
        pltpu.make_async_copy(dst_hbm.at[pl.ds(0, C)], dst0, isem0).wait()
        _drain_rows(rows0, ssem0)
        pltpu.async_copy(h_hbm.at[src0], rows0, gsem0)
        return carry
    lax.fori_loop(0, NPAIR, _pair, 0)

    pltpu.make_async_copy(h_hbm.at[pl.ds(0, C)], rows0, gsem0).wait()
    _drain_p(pden0, dsem0)
    _process(src0, dst0, rows0, scidx0, pden0, ssem0, dsem0)
    _drain_rows(rows0, ssem0)
    _drain_p(pden0, dsem0)
    _drain_rows(rows1, ssem1)
    _drain_p(pden1, dsem1)

    plsc.subcore_barrier()

    pltpu.sync_copy(acc.at[pl.ds(base, RPT8)],
                    part_hbm.at[cid, pl.ds(base, RPT8)])

    @pl.when(sid == NS - 1)
    def _tail_out():
        pltpu.sync_copy(acc.at[pl.ds(NS * RPT8, N - NS * RPT8)],
                        part_hbm.at[cid, pl.ds(NS * RPT8, N - NS * RPT8)])

    pltpu.sync_copy(den_sh.at[pl.ds(sid * ZB, ZB)],
                    den_hbm.at[cid, pl.ds(sid * ZB, ZB)])


def _sc_call(h, src, dst, a, b):
    mesh = plsc.VectorSubcoreMesh(core_axis_name="c", subcore_axis_name="s",
                                  num_cores=NC, num_subcores=NS)
    fn = pl.kernel(
        _sc_body,
        out_type=(
            jax.ShapeDtypeStruct((NC, N, D), jnp.float32),
            jax.ShapeDtypeStruct((NC, TBL), jnp.float32),
        ),
        mesh=mesh,
        compiler_params=pltpu.CompilerParams(needs_layout_passes=False,
                                             use_tc_tiling_on_sc=False),
        scratch_types=(
            pltpu.VMEM((C,), jnp.int32),
            pltpu.VMEM((C,), jnp.int32),
            pltpu.VMEM((C,), jnp.int32),
            pltpu.VMEM((C,), jnp.int32),
            pltpu.VMEM((TBL,), jnp.float32),
            pltpu.VMEM((TBL,), jnp.float32),
            pltpu.VMEM((ZB,), jnp.float32),
            pltpu.VMEM((C, D), jnp.float32),
            pltpu.VMEM((C, D), jnp.float32),
            pltpu.VMEM((C,), jnp.int32),
            pltpu.VMEM((C,), jnp.int32),
            pltpu.VMEM((C + L,), jnp.float32),
            pltpu.VMEM((C + L,), jnp.float32),
            pltpu.VMEM_SHARED((N, D), jnp.float32),
            pltpu.VMEM_SHARED((TBL,), jnp.float32),
            pltpu.SemaphoreType.DMA,
            pltpu.SemaphoreType.DMA,
            pltpu.SemaphoreType.DMA,
            pltpu.SemaphoreType.DMA,
            pltpu.SemaphoreType.DMA,
            pltpu.SemaphoreType.DMA,
            pltpu.SemaphoreType.DMA,
            pltpu.SemaphoreType.DMA,
        ),
    )
    return fn(h, src, dst, a, b)


def _fin_body(p0_ref, p1_ref, d_ref, o_ref):
    s = p0_ref[0] + p1_ref[0]
    den = jnp.sum(d_ref[...], axis=1)
    o_ref[...] = s / jnp.maximum(den, 1e-9)[:, None]


def _fin_call(partials, denoms_t):
    blk = 400
    return pl.pallas_call(
        _fin_body,
        grid=(N // blk,),
        in_specs=[
            pl.BlockSpec((1, blk, D), lambda i: (0, i, 0)),
            pl.BlockSpec((1, blk, D), lambda i: (1, i, 0)),
            pl.BlockSpec((blk, NC), lambda i: (i, 0)),
        ],
        out_specs=pl.BlockSpec((blk, D), lambda i: (i, 0)),
        out_shape=jax.ShapeDtypeStruct((N, D), jnp.float32),
    )(partials, partials, denoms_t)


@jax.jit
def kernel(h, edge_index, W_att):
    w_row = W_att[0]
    w_pad = jnp.zeros((D, 128), jnp.float32)
    w_pad = w_pad.at[:, 0].set(w_row[:D]).at[:, 1].set(w_row[D:])
    ab = _ab_call(h, w_pad)
    a = ab[:, 0]
    b = ab[:, 1]
    partials, denoms = _sc_call(h, edge_index[0], edge_index[1], a, b)
    den_t = denoms.T[:N]
    return _fin_call(partials, den_t)


# speedup vs baseline: 1.8708x; 1.0387x over previous
import functools

import jax
import jax.numpy as jnp
from jax import lax
from jax.experimental import pallas as pl
from jax.experimental.pallas import tpu as pltpu
from jax.experimental.pallas import tpu_sc as plsc

N = 10000
E = 320000
D = 128
NEG_SLOPE = 0.2

NC = 2
NS = 16
L = 16
NW = NC * NS
EW = E // NW
C = 80
NCHUNK = EW // C
NPAIR = NCHUNK // 2
RPT8 = 624
TBL = 10240
ZB = TBL // NS


def _ab_body(h_ref, w_ref, o_ref):
    o_ref[...] = jnp.dot(h_ref[...], w_ref[...],
                         preferred_element_type=jnp.float32)


def _ab_call(h, w_pad):
    blk = 1000
    return pl.pallas_call(
        _ab_body,
        grid=(N // blk,),
        in_specs=[
            pl.BlockSpec((blk, D), lambda i: (i, 0)),
            pl.BlockSpec((D, 128), lambda i: (0, 0)),
        ],
        out_specs=pl.BlockSpec((blk, 128), lambda i: (i, 0)),
        out_shape=jax.ShapeDtypeStruct((N, 128), jnp.float32),
    )(h, w_pad)


def _sc_body(h_hbm, ei_hbm, a_hbm, b_hbm, part_hbm, den_hbm,
             src0, dst0, src1, dst1, a_v, b_v, zbuf, rows0, rows1,
             scidx0, scidx1, pden0, pden1, acc, den_sh,
             gsem0, gsem1, isem0, isem1, ssem0, ssem1, dsem0, dsem1):
    cid = lax.axis_index("c")
    sid = lax.axis_index("s")
    wid = sid * NC + cid
    ebase = wid * EW

    pltpu.sync_copy(a_hbm, a_v.at[pl.ds(0, N)])
    pltpu.sync_copy(b_hbm, b_v.at[pl.ds(0, N)])

    def _zrow(i, carry):
        for j in range(D // L):
            rows0[i, pl.ds(j * L, L)] = jnp.zeros((L,), jnp.float32)
        return carry
    lax.fori_loop(0, C, _zrow, 0)

    def _zzb(i, carry):
        zbuf[pl.ds(i * L, L)] = jnp.zeros((L,), jnp.float32)
        return carry
    lax.fori_loop(0, ZB // L, _zzb, 0)

    base = sid * RPT8
    zcps = [pltpu.async_copy(rows0, acc.at[pl.ds(base + k * C, C)], gsem0)
            for k in range(RPT8 // C)]
    zcps.append(pltpu.async_copy(rows0.at[pl.ds(0, RPT8 % C)],
                                 acc.at[pl.ds(base + (RPT8 // C) * C,
                                              RPT8 % C)], gsem0))
    zcps.append(pltpu.async_copy(zbuf, den_sh.at[pl.ds(sid * ZB, ZB)], gsem0))

    @pl.when(sid == NS - 1)
    def _tail_zero():
        pltpu.sync_copy(rows0.at[pl.ds(0, N - NS * RPT8)],
                        acc.at[pl.ds(NS * RPT8, N - NS * RPT8)])
    for cp in zcps:
        cp.wait()
    plsc.subcore_barrier()

    def _process(src_r, dst_r, rows_r, scidx, pden, ssem, dsem):
        for g in range(C // L):
            sv = src_r[pl.ds(g * L, L)]
            dv = dst_r[pl.ds(g * L, L)]
            e = plsc.load_gather(a_v, [sv]) + plsc.load_gather(b_v, [dv])
            e = jnp.where(e >= 0, e, NEG_SLOPE * e)
            pden[pl.ds(g * L, L)] = jnp.exp(e)
            scidx[pl.ds(g * L, L)] = dv
        pltpu.async_copy(pden.at[pl.ds(0, C)], den_sh.at[scidx], dsem,
                         add=True)

        def _scale(i, carry2):
            pi = pden[pl.ds(i, L)][0]
            for j in range(D // L):
                rows_r[i, pl.ds(j * L, L)] = rows_r[i, pl.ds(j * L, L)] * pi
            return carry2
        lax.fori_loop(0, C, _scale, 0)

        pltpu.async_copy(rows_r, acc.at[scidx], ssem, add=True)

    pltpu.async_copy(ei_hbm.at[0, pl.ds(ebase, C)], src0, isem0).wait()
    pltpu.async_copy(ei_hbm.at[1, pl.ds(ebase, C)], dst0, isem0).wait()
    pltpu.async_copy(ei_hbm.at[0, pl.ds(ebase + C, C)], src1, isem1)
    pltpu.async_copy(ei_hbm.at[1, pl.ds(ebase + C, C)], dst1, isem1)
    pltpu.async_copy(h_hbm.at[src0], rows0, gsem0)

    def _drain_rows(rows_r, ssem):
        pltpu.make_async_copy(h_hbm.at[pl.ds(0, C)], rows_r, ssem).wait()

    def _drain_p(pden, dsem):
        pltpu.make_async_copy(a_hbm.at[pl.ds(0, C)], pden.at[pl.ds(0, C)],
                              dsem).wait()

    def _pair(k, carry):
        c0 = 2 * k

        @pl.when(k > 0)
        def _dr1():
            _drain_rows(rows1, ssem1)
            _drain_p(pden1, dsem1)

        pltpu.make_async_copy(ei_hbm.at[0, pl.ds(0, C)], src1, isem1).wait()
        pltpu.make_async_copy(ei_hbm.at[1, pl.ds(0, C)], dst1, isem1).wait()
        pltpu.async_copy(h_hbm.at[src1], rows1, gsem1)

        @pl.when(k > 0)
        def _dr0():
            _drain_p(pden0, dsem0)

        pltpu.make_async_copy(h_hbm.at[pl.ds(0, C)], rows0, gsem0).wait()
        _process(src0, dst0, rows0, scidx0, pden0, ssem0, dsem0)
        pltpu.async_copy(ei_hbm.at[0, pl.ds(ebase + (c0 + 2) * C, C)],
                         src0, isem0)
        pltpu.async_copy(ei_hbm.at[1, pl.ds(ebase + (c0 + 2) * C, C)],
                         dst0, isem0)
        pltpu.make_async_copy(h_hbm.at[pl.ds(0, C)], rows1, gsem1).wait()
        _process(src1, dst1, rows1, scidx1, pden1, ssem1, dsem1)

        @pl.when(k < NPAIR - 1)
        def _pf1():
            pltpu.async_copy(ei_hbm.at[0, pl.ds(ebase + (c0 + 3) * C, C)],
                             src1, isem1)
            pltpu.async_copy(ei_hbm.at[1, pl.ds(ebase + (c0 + 3) * C, C)],
                             dst1, isem1)
        pltpu.make_async_copy(ei_hbm.at[0, pl.ds(0, C)], src0, isem0).wait()
        pltpu.make_async_copy(ei_hbm.at[1, pl.ds(0, C)], dst0, isem0).wait()
        _drain_rows(rows0, ssem0)
        pltpu.async_copy(h_hbm.at[src0], rows0, gsem0)
        return carry
    lax.fori_loop(0, NPAIR, _pair, 0)

    pltpu.make_async_copy(h_hbm.at[pl.ds(0, C)], rows0, gsem0).wait()
    _drain_p(pden0, dsem0)
    _process(src0, dst0, rows0, scidx0, pden0, ssem0, dsem0)
    _drain_rows(rows0, ssem0)
    _drain_p(pden0, dsem0)
    _drain_rows(rows1, ssem1)
    _drain_p(pden1, dsem1)

    plsc.subcore_barrier()

    pltpu.sync_copy(acc.at[pl.ds(base, RPT8)],
                    part_hbm.at[cid, pl.ds(base, RPT8)])

    @pl.when(sid == NS - 1)
    def _tail_out():
        pltpu.sync_copy(acc.at[pl.ds(NS * RPT8, N - NS * RPT8)],
                        part_hbm.at[cid, pl.ds(NS * RPT8, N - NS * RPT8)])

    pltpu.sync_copy(den_sh.at[pl.ds(sid * ZB, ZB)],
                    den_hbm.at[cid, pl.ds(sid * ZB, ZB)])


def _sc_call(h, ei, a, b):
    mesh = plsc.VectorSubcoreMesh(core_axis_name="c", subcore_axis_name="s",
                                  num_cores=NC, num_subcores=NS)
    fn = pl.kernel(
        _sc_body,
        out_type=(
            jax.ShapeDtypeStruct((NC, N, D), jnp.float32),
            jax.ShapeDtypeStruct((NC, TBL), jnp.float32),
        ),
        mesh=mesh,
        compiler_params=pltpu.CompilerParams(needs_layout_passes=False,
                                             use_tc_tiling_on_sc=False),
        scratch_types=(
            pltpu.VMEM((C,), jnp.int32),
            pltpu.VMEM((C,), jnp.int32),
            pltpu.VMEM((C,), jnp.int32),
            pltpu.VMEM((C,), jnp.int32),
            pltpu.VMEM((TBL,), jnp.float32),
            pltpu.VMEM((TBL,), jnp.float32),
            pltpu.VMEM((ZB,), jnp.float32),
            pltpu.VMEM((C, D), jnp.float32),
            pltpu.VMEM((C, D), jnp.float32),
            pltpu.VMEM((C,), jnp.int32),
            pltpu.VMEM((C,), jnp.int32),
            pltpu.VMEM((C + L,), jnp.float32),
            pltpu.VMEM((C + L,), jnp.float32),
            pltpu.VMEM_SHARED((N, D), jnp.float32),
            pltpu.VMEM_SHARED((TBL,), jnp.float32),
            pltpu.SemaphoreType.DMA,
            pltpu.SemaphoreType.DMA,
            pltpu.SemaphoreType.DMA,
            pltpu.SemaphoreType.DMA,
            pltpu.SemaphoreType.DMA,
            pltpu.SemaphoreType.DMA,
            pltpu.SemaphoreType.DMA,
            pltpu.SemaphoreType.DMA,
        ),
    )
    return fn(h, ei, a, b)


def _fin_body(p0_ref, p1_ref, d_ref, o_ref):
    s = p0_ref[0] + p1_ref[0]
    den = jnp.sum(d_ref[...], axis=1)
    o_ref[...] = s / jnp.maximum(den, 1e-9)[:, None]


def _fin_call(partials, denoms_t):
    blk = 400
    return pl.pallas_call(
        _fin_body,
        grid=(N // blk,),
        in_specs=[
            pl.BlockSpec((1, blk, D), lambda i: (0, i, 0)),
            pl.BlockSpec((1, blk, D), lambda i: (1, i, 0)),
            pl.BlockSpec((blk, NC), lambda i: (i, 0)),
        ],
        out_specs=pl.BlockSpec((blk, D), lambda i: (i, 0)),
        out_shape=jax.ShapeDtypeStruct((N, D), jnp.float32),
    )(partials, partials, denoms_t)


@jax.jit
def kernel(h, edge_index, W_att):
    w_row = W_att[0]
    w_pad = jnp.zeros((D, 128), jnp.float32)
    w_pad = w_pad.at[:, 0].set(w_row[:D]).at[:, 1].set(w_row[D:])
    ab = _ab_call(h, w_pad)
    a = ab[:, 0]
    b = ab[:, 1]
    partials, denoms = _sc_call(h, edge_index, a, b)
    den_t = denoms.T[:N]
    return _fin_call(partials, den_t)


# speedup vs baseline: 1.9277x; 1.0304x over previous
import functools

import jax
import jax.numpy as jnp
from jax import lax
from jax.experimental import pallas as pl
from jax.experimental.pallas import tpu as pltpu
from jax.experimental.pallas import tpu_sc as plsc

N = 10000
E = 320000
D = 128
NEG_SLOPE = 0.2

NC = 2
NS = 16
L = 16
NW = NC * NS
EW = E // NW
C = 80
NCHUNK = EW // C
NPAIR = NCHUNK // 2
RPT8 = 624
TBL = 10240
ZB = TBL // NS


def _ab_body(h_ref, w_ref, o_ref):
    o_ref[...] = lax.dot_general(
        h_ref[...], w_ref[...], (((1,), (1,)), ((), ())),
        preferred_element_type=jnp.float32)


def _ab_call(h, w2):
    blk = 1000
    return pl.pallas_call(
        _ab_body,
        grid=(N // blk,),
        in_specs=[
            pl.BlockSpec((blk, D), lambda i: (i, 0)),
            pl.BlockSpec((2, D), lambda i: (0, 0)),
        ],
        out_specs=pl.BlockSpec((blk, 2), lambda i: (i, 0)),
        out_shape=jax.ShapeDtypeStruct((N, 2), jnp.float32),
    )(h, w2)


def _sc_body(h_hbm, ei_hbm, ab_hbm, part_hbm, den_hbm,
             src0, dst0, src1, dst1, ab_v, zbuf, rows0, rows1,
             scidx0, scidx1, pden0, pden1, acc, den_sh,
             gsem0, gsem1, isem0, isem1, ssem0, ssem1, dsem0, dsem1):
    cid = lax.axis_index("c")
    sid = lax.axis_index("s")
    wid = sid * NC + cid
    ebase = wid * EW

    pltpu.sync_copy(ab_hbm, ab_v.at[pl.ds(0, 2 * N)])

    def _zrow(i, carry):
        for j in range(D // L):
            rows0[i, pl.ds(j * L, L)] = jnp.zeros((L,), jnp.float32)
        return carry
    lax.fori_loop(0, C, _zrow, 0)

    def _zzb(i, carry):
        zbuf[pl.ds(i * L, L)] = jnp.zeros((L,), jnp.float32)
        return carry
    lax.fori_loop(0, ZB // L, _zzb, 0)

    base = sid * RPT8
    zcps = [pltpu.async_copy(rows0, acc.at[pl.ds(base + k * C, C)], gsem0)
            for k in range(RPT8 // C)]
    zcps.append(pltpu.async_copy(rows0.at[pl.ds(0, RPT8 % C)],
                                 acc.at[pl.ds(base + (RPT8 // C) * C,
                                              RPT8 % C)], gsem0))
    zcps.append(pltpu.async_copy(zbuf, den_sh.at[pl.ds(sid * ZB, ZB)], gsem0))

    @pl.when(sid == NS - 1)
    def _tail_zero():
        pltpu.sync_copy(rows0.at[pl.ds(0, N - NS * RPT8)],
                        acc.at[pl.ds(NS * RPT8, N - NS * RPT8)])
    for cp in zcps:
        cp.wait()
    plsc.subcore_barrier()

    def _process(src_r, dst_r, rows_r, scidx, pden, ssem, dsem):
        for g in range(C // L):
            sv = src_r[pl.ds(g * L, L)]
            dv = dst_r[pl.ds(g * L, L)]
            e = (plsc.load_gather(ab_v, [lax.shift_left(sv, 1)])
                 + plsc.load_gather(ab_v,
                                    [jnp.bitwise_or(lax.shift_left(dv, 1),
                                                    1)]))
            e = jnp.where(e >= 0, e, NEG_SLOPE * e)
            pden[pl.ds(g * L, L)] = jnp.exp(e)
            scidx[pl.ds(g * L, L)] = dv
        pltpu.async_copy(pden.at[pl.ds(0, C)], den_sh.at[scidx], dsem,
                         add=True)

        def _scale(i, carry2):
            pi = pden[pl.ds(i, L)][0]
            for j in range(D // L):
                rows_r[i, pl.ds(j * L, L)] = rows_r[i, pl.ds(j * L, L)] * pi
            return carry2
        lax.fori_loop(0, C, _scale, 0)

        pltpu.async_copy(rows_r, acc.at[scidx], ssem, add=True)

    pltpu.async_copy(ei_hbm.at[0, pl.ds(ebase, C)], src0, isem0).wait()
    pltpu.async_copy(ei_hbm.at[1, pl.ds(ebase, C)], dst0, isem0).wait()
    pltpu.async_copy(ei_hbm.at[0, pl.ds(ebase + C, C)], src1, isem1)
    pltpu.async_copy(ei_hbm.at[1, pl.ds(ebase + C, C)], dst1, isem1)
    pltpu.async_copy(h_hbm.at[src0], rows0, gsem0)

    def _drain_rows(rows_r, ssem):
        pltpu.make_async_copy(h_hbm.at[pl.ds(0, C)], rows_r, ssem).wait()

    def _drain_p(pden, dsem):
        pltpu.make_async_copy(ab_hbm.at[pl.ds(0, C)], pden.at[pl.ds(0, C)],
                              dsem).wait()

    def _pair(k, carry):
        c0 = 2 * k

        @pl.when(k > 0)
        def _dr1():
            _drain_rows(rows1, ssem1)
            _drain_p(pden1, dsem1)

        pltpu.make_async_copy(ei_hbm.at[0, pl.ds(0, C)], src1, isem1).wait()
        pltpu.make_async_copy(ei_hbm.at[1, pl.ds(0, C)], dst1, isem1).wait()
        pltpu.async_copy(h_hbm.at[src1], rows1, gsem1)

        @pl.when(k > 0)
        def _dr0():
            _drain_p(pden0, dsem0)

        pltpu.make_async_copy(h_hbm.at[pl.ds(0, C)], rows0, gsem0).wait()
        _process(src0, dst0, rows0, scidx0, pden0, ssem0, dsem0)
        pltpu.async_copy(ei_hbm.at[0, pl.ds(ebase + (c0 + 2) * C, C)],
                         src0, isem0)
        pltpu.async_copy(ei_hbm.at[1, pl.ds(ebase + (c0 + 2) * C, C)],
                         dst0, isem0)
        pltpu.make_async_copy(h_hbm.at[pl.ds(0, C)], rows1, gsem1).wait()
        _process(src1, dst1, rows1, scidx1, pden1, ssem1, dsem1)

        @pl.when(k < NPAIR - 1)
        def _pf1():
            pltpu.async_copy(ei_hbm.at[0, pl.ds(ebase + (c0 + 3) * C, C)],
                             src1, isem1)
            pltpu.async_copy(ei_hbm.at[1, pl.ds(ebase + (c0 + 3) * C, C)],
                             dst1, isem1)
        pltpu.make_async_copy(ei_hbm.at[0, pl.ds(0, C)], src0, isem0).wait()
        pltpu.make_async_copy(ei_hbm.at[1, pl.ds(0, C)], dst0, isem0).wait()
        _drain_rows(rows0, ssem0)
        pltpu.async_copy(h_hbm.at[src0], rows0, gsem0)
        return carry
    lax.fori_loop(0, NPAIR, _pair, 0)

    pltpu.make_async_copy(h_hbm.at[pl.ds(0, C)], rows0, gsem0).wait()
    _drain_p(pden0, dsem0)
    _process(src0, dst0, rows0, scidx0, pden0, ssem0, dsem0)
    _drain_rows(rows0, ssem0)
    _drain_p(pden0, dsem0)
    _drain_rows(rows1, ssem1)
    _drain_p(pden1, dsem1)

    plsc.subcore_barrier()

    pltpu.sync_copy(acc.at[pl.ds(base, RPT8)],
                    part_hbm.at[cid, pl.ds(base, RPT8)])

    @pl.when(sid == NS - 1)
    def _tail_out():
        pltpu.sync_copy(acc.at[pl.ds(NS * RPT8, N - NS * RPT8)],
                        part_hbm.at[cid, pl.ds(NS * RPT8, N - NS * RPT8)])

    pltpu.sync_copy(den_sh.at[pl.ds(sid * ZB, ZB)],
                    den_hbm.at[cid, pl.ds(sid * ZB, ZB)])


def _sc_call(h, ei, ab_flat):
    mesh = plsc.VectorSubcoreMesh(core_axis_name="c", subcore_axis_name="s",
                                  num_cores=NC, num_subcores=NS)
    fn = pl.kernel(
        _sc_body,
        out_type=(
            jax.ShapeDtypeStruct((NC, N, D), jnp.float32),
            jax.ShapeDtypeStruct((NC, TBL), jnp.float32),
        ),
        mesh=mesh,
        compiler_params=pltpu.CompilerParams(needs_layout_passes=False,
                                             use_tc_tiling_on_sc=False),
        scratch_types=(
            pltpu.VMEM((C,), jnp.int32),
            pltpu.VMEM((C,), jnp.int32),
            pltpu.VMEM((C,), jnp.int32),
            pltpu.VMEM((C,), jnp.int32),
            pltpu.VMEM((2 * TBL,), jnp.float32),
            pltpu.VMEM((ZB,), jnp.float32),
            pltpu.VMEM((C, D), jnp.float32),
            pltpu.VMEM((C, D), jnp.float32),
            pltpu.VMEM((C,), jnp.int32),
            pltpu.VMEM((C,), jnp.int32),
            pltpu.VMEM((C + L,), jnp.float32),
            pltpu.VMEM((C + L,), jnp.float32),
            pltpu.VMEM_SHARED((N, D), jnp.float32),
            pltpu.VMEM_SHARED((TBL,), jnp.float32),
            pltpu.SemaphoreType.DMA,
            pltpu.SemaphoreType.DMA,
            pltpu.SemaphoreType.DMA,
            pltpu.SemaphoreType.DMA,
            pltpu.SemaphoreType.DMA,
            pltpu.SemaphoreType.DMA,
            pltpu.SemaphoreType.DMA,
            pltpu.SemaphoreType.DMA,
        ),
    )
    return fn(h, ei, ab_flat)


def _fin_body(p0_ref, p1_ref, d_ref, o_ref):
    s = p0_ref[0] + p1_ref[0]
    den = jnp.sum(d_ref[...], axis=1)
    o_ref[...] = s / jnp.maximum(den, 1e-9)[:, None]


def _fin_call(partials, denoms_t):
    blk = 400
    return pl.pallas_call(
        _fin_body,
        grid=(N // blk,),
        in_specs=[
            pl.BlockSpec((1, blk, D), lambda i: (0, i, 0)),
            pl.BlockSpec((1, blk, D), lambda i: (1, i, 0)),
            pl.BlockSpec((blk, NC), lambda i: (i, 0)),
        ],
        out_specs=pl.BlockSpec((blk, D), lambda i: (i, 0)),
        out_shape=jax.ShapeDtypeStruct((N, D), jnp.float32),
    )(partials, partials, denoms_t)


@jax.jit
def kernel(h, edge_index, W_att):
    ab = _ab_call(h, W_att.reshape(2, D))
    partials, denoms = _sc_call(h, edge_index, ab.reshape(2 * N))
    den_t = denoms.T[:N]
    return _fin_call(partials, den_t)


# speedup vs baseline: 2.1891x; 1.1356x over previous
import functools

import jax
import jax.numpy as jnp
from jax import lax
from jax.experimental import pallas as pl
from jax.experimental.pallas import tpu as pltpu
from jax.experimental.pallas import tpu_sc as plsc

N = 10000
E = 320000
D = 128
NEG_SLOPE = 0.2

NC = 2
NS = 16
L = 16
NW = NC * NS
EW = E // NW
C = 80
NCHUNK = EW // C
NPAIR = NCHUNK // 2
RPT8 = 624
TBL = 10240
ZB = TBL // NS


def _ab_body(h_ref, w_ref, o_ref):
    o_ref[...] = lax.dot_general(
        h_ref[...], w_ref[...], (((1,), (1,)), ((), ())),
        preferred_element_type=jnp.float32)


def _ab_call(h, w2):
    blk = 1000
    return pl.pallas_call(
        _ab_body,
        grid=(N // blk,),
        in_specs=[
            pl.BlockSpec((blk, D), lambda i: (i, 0)),
            pl.BlockSpec((2, D), lambda i: (0, 0)),
        ],
        out_specs=pl.BlockSpec((blk, 2), lambda i: (i, 0)),
        out_shape=jax.ShapeDtypeStruct((N, 2), jnp.float32),
    )(h, w2)


def _sc_body(h_hbm, ei_hbm, ab_hbm, part_hbm, den_hbm,
             src0, dst0, src1, dst1, ab_v, zbuf, rows0, rows1,
             scidx0, scidx1, pden0, pden1, acc, den_sh,
             gsem0, gsem1, isem0, isem1, ssem0, ssem1, dsem0, dsem1):
    cid = lax.axis_index("c")
    sid = lax.axis_index("s")
    wid = sid * NC + cid
    ebase = wid * EW

    pltpu.sync_copy(ab_hbm, ab_v.at[pl.ds(0, 2 * N)])

    def _zrow(i, carry):
        for j in range(D // L):
            rows0[i, pl.ds(j * L, L)] = jnp.zeros((L,), jnp.float32)
        return carry
    lax.fori_loop(0, C, _zrow, 0)

    def _zzb(i, carry):
        zbuf[pl.ds(i * L, L)] = jnp.zeros((L,), jnp.float32)
        return carry
    lax.fori_loop(0, ZB // L, _zzb, 0)

    base = sid * RPT8
    zcps = [pltpu.async_copy(rows0, acc.at[pl.ds(base + k * C, C)], gsem0)
            for k in range(RPT8 // C)]
    zcps.append(pltpu.async_copy(rows0.at[pl.ds(0, RPT8 % C)],
                                 acc.at[pl.ds(base + (RPT8 // C) * C,
                                              RPT8 % C)], gsem0))
    zcps.append(pltpu.async_copy(zbuf, den_sh.at[pl.ds(sid * ZB, ZB)], gsem0))

    @pl.when(sid == NS - 1)
    def _tail_zero():
        pltpu.sync_copy(rows0.at[pl.ds(0, N - NS * RPT8)],
                        acc.at[pl.ds(NS * RPT8, N - NS * RPT8)])
    for cp in zcps:
        cp.wait()
    plsc.subcore_barrier()

    def _process(src_r, dst_r, rows_r, scidx, pden, ssem, dsem):
        for g in range(C // L):
            sv = src_r[pl.ds(g * L, L)]
            dv = dst_r[pl.ds(g * L, L)]
            e = (plsc.load_gather(ab_v, [lax.shift_left(sv, 1)])
                 + plsc.load_gather(ab_v,
                                    [jnp.bitwise_or(lax.shift_left(dv, 1),
                                                    1)]))
            e = jnp.where(e >= 0, e, NEG_SLOPE * e)
            pden[pl.ds(g * L, L)] = jnp.exp(e)
            scidx[pl.ds(g * L, L)] = dv
        pltpu.async_copy(pden.at[pl.ds(0, C)], den_sh.at[scidx], dsem,
                         add=True)

        def _scale(i2, carry2):
            i = 2 * i2
            pi0 = pden[pl.ds(i, L)][0]
            pi1 = pden[pl.ds(i + 1, L)][0]
            for j in range(D // L):
                rows_r[i, pl.ds(j * L, L)] = rows_r[i, pl.ds(j * L, L)] * pi0
            for j in range(D // L):
                rows_r[i + 1, pl.ds(j * L, L)] = (
                    rows_r[i + 1, pl.ds(j * L, L)] * pi1)
            return carry2
        lax.fori_loop(0, C // 2, _scale, 0)

        pltpu.async_copy(rows_r, acc.at[scidx], ssem, add=True)

    pltpu.async_copy(ei_hbm.at[0, pl.ds(ebase, C)], src0, isem0).wait()
    pltpu.async_copy(ei_hbm.at[1, pl.ds(ebase, C)], dst0, isem0).wait()
    pltpu.async_copy(ei_hbm.at[0, pl.ds(ebase + C, C)], src1, isem1)
    pltpu.async_copy(ei_hbm.at[1, pl.ds(ebase + C, C)], dst1, isem1)
    pltpu.async_copy(h_hbm.at[src0], rows0, gsem0)

    def _drain_rows(rows_r, ssem):
        pltpu.make_async_copy(h_hbm.at[pl.ds(0, C)], rows_r, ssem).wait()

    def _drain_p(pden, dsem):
        pltpu.make_async_copy(ab_hbm.at[pl.ds(0, C)], pden.at[pl.ds(0, C)],
                              dsem).wait()

    def _pair(k, carry):
        c0 = 2 * k

        @pl.when(k > 0)
        def _dr1():
            _drain_rows(rows1, ssem1)
            _drain_p(pden1, dsem1)

        pltpu.make_async_copy(ei_hbm.at[0, pl.ds(0, C)], src1, isem1).wait()
        pltpu.make_async_copy(ei_hbm.at[1, pl.ds(0, C)], dst1, isem1).wait()
        pltpu.async_copy(h_hbm.at[src1], rows1, gsem1)

        @pl.when(k > 0)
        def _dr0():
            _drain_p(pden0, dsem0)

        pltpu.make_async_copy(h_hbm.at[pl.ds(0, C)], rows0, gsem0).wait()
        _process(src0, dst0, rows0, scidx0, pden0, ssem0, dsem0)
        pltpu.async_copy(ei_hbm.at[0, pl.ds(ebase + (c0 + 2) * C, C)],
                         src0, isem0)
        pltpu.async_copy(ei_hbm.at[1, pl.ds(ebase + (c0 + 2) * C, C)],
                         dst0, isem0)
        pltpu.make_async_copy(h_hbm.at[pl.ds(0, C)], rows1, gsem1).wait()
        _process(src1, dst1, rows1, scidx1, pden1, ssem1, dsem1)

        @pl.when(k < NPAIR - 1)
        def _pf1():
            pltpu.async_copy(ei_hbm.at[0, pl.ds(ebase + (c0 + 3) * C, C)],
                             src1, isem1)
            pltpu.async_copy(ei_hbm.at[1, pl.ds(ebase + (c0 + 3) * C, C)],
                             dst1, isem1)
        pltpu.make_async_copy(ei_hbm.at[0, pl.ds(0, C)], src0, isem0).wait()
        pltpu.make_async_copy(ei_hbm.at[1, pl.ds(0, C)], dst0, isem0).wait()
        _drain_rows(rows0, ssem0)
        pltpu.async_copy(h_hbm.at[src0], rows0, gsem0)
        return carry
    lax.fori_loop(0, NPAIR, _pair, 0)

    pltpu.make_async_copy(h_hbm.at[pl.ds(0, C)], rows0, gsem0).wait()
    _drain_p(pden0, dsem0)
    _process(src0, dst0, rows0, scidx0, pden0, ssem0, dsem0)
    _drain_rows(rows0, ssem0)
    _drain_p(pden0, dsem0)
    _drain_rows(rows1, ssem1)
    _drain_p(pden1, dsem1)

    plsc.subcore_barrier()

    pltpu.sync_copy(acc.at[pl.ds(base, RPT8)],
                    part_hbm.at[cid, pl.ds(base, RPT8)])

    @pl.when(sid == NS - 1)
    def _tail_out():
        pltpu.sync_copy(acc.at[pl.ds(NS * RPT8, N - NS * RPT8)],
                        part_hbm.at[cid, pl.ds(NS * RPT8, N - NS * RPT8)])

    pltpu.sync_copy(den_sh.at[pl.ds(sid * ZB, ZB)],
                    den_hbm.at[cid, pl.ds(sid * ZB, ZB)])


def _sc_call(h, ei, ab_flat):
    mesh = plsc.VectorSubcoreMesh(core_axis_name="c", subcore_axis_name="s",
                                  num_cores=NC, num_subcores=NS)
    fn = pl.kernel(
        _sc_body,
        out_type=(
            jax.ShapeDtypeStruct((NC, N, D), jnp.float32),
            jax.ShapeDtypeStruct((NC, TBL), jnp.float32),
        ),
        mesh=mesh,
        compiler_params=pltpu.CompilerParams(needs_layout_passes=False,
                                             use_tc_tiling_on_sc=False),
        scratch_types=(
            pltpu.VMEM((C,), jnp.int32),
            pltpu.VMEM((C,), jnp.int32),
            pltpu.VMEM((C,), jnp.int32),
            pltpu.VMEM((C,), jnp.int32),
            pltpu.VMEM((2 * TBL,), jnp.float32),
            pltpu.VMEM((ZB,), jnp.float32),
            pltpu.VMEM((C, D), jnp.float32),
            pltpu.VMEM((C, D), jnp.float32),
            pltpu.VMEM((C,), jnp.int32),
            pltpu.VMEM((C,), jnp.int32),
            pltpu.VMEM((C + L,), jnp.float32),
            pltpu.VMEM((C + L,), jnp.float32),
            pltpu.VMEM_SHARED((N, D), jnp.float32),
            pltpu.VMEM_SHARED((TBL,), jnp.float32),
            pltpu.SemaphoreType.DMA,
            pltpu.SemaphoreType.DMA,
            pltpu.SemaphoreType.DMA,
            pltpu.SemaphoreType.DMA,
            pltpu.SemaphoreType.DMA,
            pltpu.SemaphoreType.DMA,
            pltpu.SemaphoreType.DMA,
            pltpu.SemaphoreType.DMA,
        ),
    )
    return fn(h, ei, ab_flat)


def _fin_body(p0_ref, p1_ref, d_ref, o_ref):
    s = p0_ref[0] + p1_ref[0]
    den = jnp.sum(d_ref[...], axis=1)
    o_ref[...] = s / jnp.maximum(den, 1e-9)[:, None]


def _fin_call(partials, denoms_t):
    blk = 400
    return pl.pallas_call(
        _fin_body,
        grid=(N // blk,),
        in_specs=[
            pl.BlockSpec((1, blk, D), lambda i: (0, i, 0)),
            pl.BlockSpec((1, blk, D), lambda i: (1, i, 0)),
            pl.BlockSpec((blk, NC), lambda i: (i, 0)),
        ],
        out_specs=pl.BlockSpec((blk, D), lambda i: (i, 0)),
        out_shape=jax.ShapeDtypeStruct((N, D), jnp.float32),
    )(partials, partials, denoms_t)


@jax.jit
def kernel(h, edge_index, W_att):
    ab = _ab_call(h, W_att.reshape(2, D))
    partials, denoms = _sc_call(h, edge_index, ab.reshape(2 * N))
    den_t = denoms.T[:N]
    return _fin_call(partials, den_t)


# speedup vs baseline: 2.2734x; 1.0385x over previous
import functools

import jax
import jax.numpy as jnp
from jax import lax
from jax.experimental import pallas as pl
from jax.experimental.pallas import tpu as pltpu
from jax.experimental.pallas import tpu_sc as plsc

N = 10000
E = 320000
D = 128
NEG_SLOPE = 0.2

NC = 2
NS = 16
L = 16
NW = NC * NS
EW = E // NW
C = 80
NCHUNK = EW // C
NPAIR = NCHUNK // 2
RPT8 = 624
TBL = 10240
ZB = TBL // NS


def _ab_body(h_ref, w_ref, o_ref):
    o_ref[...] = lax.dot_general(
        h_ref[...], w_ref[...], (((1,), (1,)), ((), ())),
        preferred_element_type=jnp.float32)


def _ab_call(h, w2):
    blk = 1000
    return pl.pallas_call(
        _ab_body,
        grid=(N // blk,),
        in_specs=[
            pl.BlockSpec((blk, D), lambda i: (i, 0)),
            pl.BlockSpec((2, D), lambda i: (0, 0)),
        ],
        out_specs=pl.BlockSpec((blk, 2), lambda i: (i, 0)),
        out_shape=jax.ShapeDtypeStruct((N, 2), jnp.float32),
    )(h, w2)


def _sc_body(h_hbm, ei_hbm, ab_hbm, part_hbm, den_hbm,
             src0, dst0, src1, dst1, ab_v, zbuf, rows0, rows1,
             scidx0, scidx1, pden0, pden1, acc, den_sh,
             gsem0, gsem1, isem0, isem1, ssem0, ssem1, dsem0, dsem1):
    cid = lax.axis_index("c")
    sid = lax.axis_index("s")
    wid = sid * NC + cid
    ebase = wid * EW

    pltpu.sync_copy(ab_hbm, ab_v.at[pl.ds(0, 2 * N)])

    def _zrow(i, carry):
        for j in range(D // L):
            rows0[i, pl.ds(j * L, L)] = jnp.zeros((L,), jnp.float32)
        return carry
    lax.fori_loop(0, C, _zrow, 0)

    def _zzb(i, carry):
        zbuf[pl.ds(i * L, L)] = jnp.zeros((L,), jnp.float32)
        return carry
    lax.fori_loop(0, ZB // L, _zzb, 0)

    base = sid * RPT8
    zcps = [pltpu.async_copy(rows0, acc.at[pl.ds(base + k * C, C)], gsem0)
            for k in range(RPT8 // C)]
    zcps.append(pltpu.async_copy(rows0.at[pl.ds(0, RPT8 % C)],
                                 acc.at[pl.ds(base + (RPT8 // C) * C,
                                              RPT8 % C)], gsem0))
    zcps.append(pltpu.async_copy(zbuf, den_sh.at[pl.ds(sid * ZB, ZB)], gsem0))

    @pl.when(sid == NS - 1)
    def _tail_zero():
        pltpu.sync_copy(rows0.at[pl.ds(0, N - NS * RPT8)],
                        acc.at[pl.ds(NS * RPT8, N - NS * RPT8)])
    for cp in zcps:
        cp.wait()
    plsc.subcore_barrier()

    def _process(src_r, dst_r, rows_r, scidx, pden, ssem, dsem):
        for g in range(C // L):
            sv = src_r[pl.ds(g * L, L)]
            dv = dst_r[pl.ds(g * L, L)]
            e = (plsc.load_gather(ab_v, [lax.shift_left(sv, 1)])
                 + plsc.load_gather(ab_v,
                                    [jnp.bitwise_or(lax.shift_left(dv, 1),
                                                    1)]))
            e = jnp.where(e >= 0, e, NEG_SLOPE * e)
            pden[pl.ds(g * L, L)] = jnp.exp(e)
            scidx[pl.ds(g * L, L)] = dv
        pltpu.async_copy(pden.at[pl.ds(0, C)], den_sh.at[scidx], dsem,
                         add=True)

        def _scale(i4, carry2):
            i = 4 * i4
            pis = [pden[pl.ds(i + u, L)][0] for u in range(4)]
            for u in range(4):
                for j in range(D // L):
                    rows_r[i + u, pl.ds(j * L, L)] = (
                        rows_r[i + u, pl.ds(j * L, L)] * pis[u])
            return carry2
        lax.fori_loop(0, C // 4, _scale, 0)

        pltpu.async_copy(rows_r, acc.at[scidx], ssem, add=True)

    pltpu.async_copy(ei_hbm.at[0, pl.ds(ebase, C)], src0, isem0).wait()
    pltpu.async_copy(ei_hbm.at[1, pl.ds(ebase, C)], dst0, isem0).wait()
    pltpu.async_copy(ei_hbm.at[0, pl.ds(ebase + C, C)], src1, isem1)
    pltpu.async_copy(ei_hbm.at[1, pl.ds(ebase + C, C)], dst1, isem1)
    pltpu.async_copy(h_hbm.at[src0], rows0, gsem0)

    def _drain_rows(rows_r, ssem):
        pltpu.make_async_copy(h_hbm.at[pl.ds(0, C)], rows_r, ssem).wait()

    def _drain_p(pden, dsem):
        pltpu.make_async_copy(ab_hbm.at[pl.ds(0, C)], pden.at[pl.ds(0, C)],
                              dsem).wait()

    def _pair(k, carry):
        c0 = 2 * k

        @pl.when(k > 0)
        def _dr1():
            _drain_rows(rows1, ssem1)
            _drain_p(pden1, dsem1)

        pltpu.make_async_copy(ei_hbm.at[0, pl.ds(0, C)], src1, isem1).wait()
        pltpu.make_async_copy(ei_hbm.at[1, pl.ds(0, C)], dst1, isem1).wait()
        pltpu.async_copy(h_hbm.at[src1], rows1, gsem1)

        @pl.when(k > 0)
        def _dr0():
            _drain_p(pden0, dsem0)

        pltpu.make_async_copy(h_hbm.at[pl.ds(0, C)], rows0, gsem0).wait()
        _process(src0, dst0, rows0, scidx0, pden0, ssem0, dsem0)
        pltpu.async_copy(ei_hbm.at[0, pl.ds(ebase + (c0 + 2) * C, C)],
                         src0, isem0)
        pltpu.async_copy(ei_hbm.at[1, pl.ds(ebase + (c0 + 2) * C, C)],
                         dst0, isem0)
        pltpu.make_async_copy(h_hbm.at[pl.ds(0, C)], rows1, gsem1).wait()
        _process(src1, dst1, rows1, scidx1, pden1, ssem1, dsem1)

        @pl.when(k < NPAIR - 1)
        def _pf1():
            pltpu.async_copy(ei_hbm.at[0, pl.ds(ebase + (c0 + 3) * C, C)],
                             src1, isem1)
            pltpu.async_copy(ei_hbm.at[1, pl.ds(ebase + (c0 + 3) * C, C)],
                             dst1, isem1)
        pltpu.make_async_copy(ei_hbm.at[0, pl.ds(0, C)], src0, isem0).wait()
        pltpu.make_async_copy(ei_hbm.at[1, pl.ds(0, C)], dst0, isem0).wait()
        _drain_rows(rows0, ssem0)
        pltpu.async_copy(h_hbm.at[src0], rows0, gsem0)
        return carry
    lax.fori_loop(0, NPAIR, _pair, 0)

    pltpu.make_async_copy(h_hbm.at[pl.ds(0, C)], rows0, gsem0).wait()
    _drain_p(pden0, dsem0)
    _process(src0, dst0, rows0, scidx0, pden0, ssem0, dsem0)
    _drain_rows(rows0, ssem0)
    _drain_p(pden0, dsem0)
    _drain_rows(rows1, ssem1)
    _drain_p(pden1, dsem1)

    plsc.subcore_barrier()

    pltpu.sync_copy(acc.at[pl.ds(base, RPT8)],
                    part_hbm.at[cid, pl.ds(base, RPT8)])

    @pl.when(sid == NS - 1)
    def _tail_out():
        pltpu.sync_copy(acc.at[pl.ds(NS * RPT8, N - NS * RPT8)],
                        part_hbm.at[cid, pl.ds(NS * RPT8, N - NS * RPT8)])

    pltpu.sync_copy(den_sh.at[pl.ds(sid * ZB, ZB)],
                    den_hbm.at[cid, pl.ds(sid * ZB, ZB)])


def _sc_call(h, ei, ab_flat):
    mesh = plsc.VectorSubcoreMesh(core_axis_name="c", subcore_axis_name="s",
                                  num_cores=NC, num_subcores=NS)
    fn = pl.kernel(
        _sc_body,
        out_type=(
            jax.ShapeDtypeStruct((NC, N, D), jnp.float32),
            jax.ShapeDtypeStruct((NC, TBL), jnp.float32),
        ),
        mesh=mesh,
        compiler_params=pltpu.CompilerParams(needs_layout_passes=False,
                                             use_tc_tiling_on_sc=False),
        scratch_types=(
            pltpu.VMEM((C,), jnp.int32),
            pltpu.VMEM((C,), jnp.int32),
            pltpu.VMEM((C,), jnp.int32),
            pltpu.VMEM((C,), jnp.int32),
            pltpu.VMEM((2 * TBL,), jnp.float32),
            pltpu.VMEM((ZB,), jnp.float32),
            pltpu.VMEM((C, D), jnp.float32),
            pltpu.VMEM((C, D), jnp.float32),
            pltpu.VMEM((C,), jnp.int32),
            pltpu.VMEM((C,), jnp.int32),
            pltpu.VMEM((C + L,), jnp.float32),
            pltpu.VMEM((C + L,), jnp.float32),
            pltpu.VMEM_SHARED((N, D), jnp.float32),
            pltpu.VMEM_SHARED((TBL,), jnp.float32),
            pltpu.SemaphoreType.DMA,
            pltpu.SemaphoreType.DMA,
            pltpu.SemaphoreType.DMA,
            pltpu.SemaphoreType.DMA,
            pltpu.SemaphoreType.DMA,
            pltpu.SemaphoreType.DMA,
            pltpu.SemaphoreType.DMA,
            pltpu.SemaphoreType.DMA,
        ),
    )
    return fn(h, ei, ab_flat)


def _fin_body(p0_ref, p1_ref, d_ref, o_ref):
    s = p0_ref[0] + p1_ref[0]
    den = jnp.sum(d_ref[...], axis=1)
    o_ref[...] = s / jnp.maximum(den, 1e-9)[:, None]


def _fin_call(partials, denoms_t):
    blk = 400
    return pl.pallas_call(
        _fin_body,
        grid=(N // blk,),
        in_specs=[
            pl.BlockSpec((1, blk, D), lambda i: (0, i, 0)),
            pl.BlockSpec((1, blk, D), lambda i: (1, i, 0)),
            pl.BlockSpec((blk, NC), lambda i: (i, 0)),
        ],
        out_specs=pl.BlockSpec((blk, D), lambda i: (i, 0)),
        out_shape=jax.ShapeDtypeStruct((N, D), jnp.float32),
    )(partials, partials, denoms_t)


@jax.jit
def kernel(h, edge_index, W_att):
    ab = _ab_call(h, W_att.reshape(2, D))
    partials, denoms = _sc_call(h, edge_index, ab.reshape(2 * N))
    den_t = denoms.T[:N]
    return _fin_call(partials, den_t)


# speedup vs baseline: 2.2871x; 1.0060x over previous
import functools

import jax
import jax.numpy as jnp
from jax import lax
from jax.experimental import pallas as pl
from jax.experimental.pallas import tpu as pltpu
from jax.experimental.pallas import tpu_sc as plsc

N = 10000
E = 320000
D = 128
NEG_SLOPE = 0.2

NC = 2
NS = 16
L = 16
NW = NC * NS
EW = E // NW
C = 80
NCHUNK = EW // C
NPAIR = NCHUNK // 2
RPT8 = 624
TBL = 10240
ZB = TBL // NS


def _ab_body(h_ref, w_ref, o_ref):
    o_ref[...] = lax.dot_general(
        h_ref[...], w_ref[...], (((1,), (1,)), ((), ())),
        preferred_element_type=jnp.float32)


def _ab_call(h, w2):
    blk = 1000
    return pl.pallas_call(
        _ab_body,
        grid=(N // blk,),
        in_specs=[
            pl.BlockSpec((blk, D), lambda i: (i, 0)),
            pl.BlockSpec((2, D), lambda i: (0, 0)),
        ],
        out_specs=pl.BlockSpec((blk, 2), lambda i: (i, 0)),
        out_shape=jax.ShapeDtypeStruct((N, 2), jnp.float32),
    )(h, w2)


def _sc_body(h_hbm, ei_hbm, ab_hbm, part_hbm, den_hbm,
             src0, dst0, src1, dst1, ab_v, zbuf, rows0, rows1,
             scidx0, scidx1, pden0, pden1, acc, den_sh,
             gsem0, gsem1, isem0, isem1, ssem0, ssem1, dsem0, dsem1):
    cid = lax.axis_index("c")
    sid = lax.axis_index("s")
    wid = sid * NC + cid
    ebase = wid * EW

    pltpu.sync_copy(ab_hbm, ab_v.at[pl.ds(0, 2 * N)])

    def _zrow(i, carry):
        for j in range(D // L):
            rows0[i, pl.ds(j * L, L)] = jnp.zeros((L,), jnp.float32)
        return carry
    lax.fori_loop(0, C, _zrow, 0)

    def _zzb(i, carry):
        zbuf[pl.ds(i * L, L)] = jnp.zeros((L,), jnp.float32)
        return carry
    lax.fori_loop(0, ZB // L, _zzb, 0)

    base = sid * RPT8
    zcps = [pltpu.async_copy(rows0, acc.at[pl.ds(base + k * C, C)], gsem0)
            for k in range(RPT8 // C)]
    zcps.append(pltpu.async_copy(rows0.at[pl.ds(0, RPT8 % C)],
                                 acc.at[pl.ds(base + (RPT8 // C) * C,
                                              RPT8 % C)], gsem0))
    zcps.append(pltpu.async_copy(zbuf, den_sh.at[pl.ds(sid * ZB, ZB)], gsem0))

    @pl.when(sid == NS - 1)
    def _tail_zero():
        pltpu.sync_copy(rows0.at[pl.ds(0, N - NS * RPT8)],
                        acc.at[pl.ds(NS * RPT8, N - NS * RPT8)])
    for cp in zcps:
        cp.wait()
    plsc.subcore_barrier()

    def _process(src_r, dst_r, rows_r, scidx, pden, ssem, dsem):
        for g in range(C // L):
            sv = src_r[pl.ds(g * L, L)]
            dv = dst_r[pl.ds(g * L, L)]
            e = (plsc.load_gather(ab_v, [lax.shift_left(sv, 1)])
                 + plsc.load_gather(ab_v,
                                    [jnp.bitwise_or(lax.shift_left(dv, 1),
                                                    1)]))
            e = jnp.where(e >= 0, e, NEG_SLOPE * e)
            pden[pl.ds(g * L, L)] = jnp.exp(e)
            scidx[pl.ds(g * L, L)] = dv
        pltpu.async_copy(pden.at[pl.ds(0, C)], den_sh.at[scidx], dsem,
                         add=True)

        def _scale(i8, carry2):
            i = 8 * i8
            pis = [pden[pl.ds(i + u, L)][0] for u in range(8)]
            for u in range(8):
                for j in range(D // L):
                    rows_r[i + u, pl.ds(j * L, L)] = (
                        rows_r[i + u, pl.ds(j * L, L)] * pis[u])
            return carry2
        lax.fori_loop(0, C // 8, _scale, 0)

        pltpu.async_copy(rows_r, acc.at[scidx], ssem, add=True)

    pltpu.async_copy(ei_hbm.at[0, pl.ds(ebase, C)], src0, isem0).wait()
    pltpu.async_copy(ei_hbm.at[1, pl.ds(ebase, C)], dst0, isem0).wait()
    pltpu.async_copy(ei_hbm.at[0, pl.ds(ebase + C, C)], src1, isem1)
    pltpu.async_copy(ei_hbm.at[1, pl.ds(ebase + C, C)], dst1, isem1)
    pltpu.async_copy(h_hbm.at[src0], rows0, gsem0)

    def _drain_rows(rows_r, ssem):
        pltpu.make_async_copy(h_hbm.at[pl.ds(0, C)], rows_r, ssem).wait()

    def _drain_p(pden, dsem):
        pltpu.make_async_copy(ab_hbm.at[pl.ds(0, C)], pden.at[pl.ds(0, C)],
                              dsem).wait()

    def _pair(k, carry):
        c0 = 2 * k

        @pl.when(k > 0)
        def _dr1():
            _drain_rows(rows1, ssem1)
            _drain_p(pden1, dsem1)

        pltpu.make_async_copy(ei_hbm.at[0, pl.ds(0, C)], src1, isem1).wait()
        pltpu.make_async_copy(ei_hbm.at[1, pl.ds(0, C)], dst1, isem1).wait()
        pltpu.async_copy(h_hbm.at[src1], rows1, gsem1)

        @pl.when(k > 0)
        def _dr0():
            _drain_p(pden0, dsem0)

        pltpu.make_async_copy(h_hbm.at[pl.ds(0, C)], rows0, gsem0).wait()
        _process(src0, dst0, rows0, scidx0, pden0, ssem0, dsem0)
        pltpu.async_copy(ei_hbm.at[0, pl.ds(ebase + (c0 + 2) * C, C)],
                         src0, isem0)
        pltpu.async_copy(ei_hbm.at[1, pl.ds(ebase + (c0 + 2) * C, C)],
                         dst0, isem0)
        pltpu.make_async_copy(h_hbm.at[pl.ds(0, C)], rows1, gsem1).wait()
        _process(src1, dst1, rows1, scidx1, pden1, ssem1, dsem1)

        @pl.when(k < NPAIR - 1)
        def _pf1():
            pltpu.async_copy(ei_hbm.at[0, pl.ds(ebase + (c0 + 3) * C, C)],
                             src1, isem1)
            pltpu.async_copy(ei_hbm.at[1, pl.ds(ebase + (c0 + 3) * C, C)],
                             dst1, isem1)
        pltpu.make_async_copy(ei_hbm.at[0, pl.ds(0, C)], src0, isem0).wait()
        pltpu.make_async_copy(ei_hbm.at[1, pl.ds(0, C)], dst0, isem0).wait()
        _drain_rows(rows0, ssem0)
        pltpu.async_copy(h_hbm.at[src0], rows0, gsem0)
        return carry
    lax.fori_loop(0, NPAIR, _pair, 0)

    pltpu.make_async_copy(h_hbm.at[pl.ds(0, C)], rows0, gsem0).wait()
    _drain_p(pden0, dsem0)
    _process(src0, dst0, rows0, scidx0, pden0, ssem0, dsem0)
    _drain_rows(rows0, ssem0)
    _drain_p(pden0, dsem0)
    _drain_rows(rows1, ssem1)
    _drain_p(pden1, dsem1)

    plsc.subcore_barrier()

    pltpu.sync_copy(acc.at[pl.ds(base, RPT8)],
                    part_hbm.at[cid, pl.ds(base, RPT8)])

    @pl.when(sid == NS - 1)
    def _tail_out():
        pltpu.sync_copy(acc.at[pl.ds(NS * RPT8, N - NS * RPT8)],
                        part_hbm.at[cid, pl.ds(NS * RPT8, N - NS * RPT8)])

    pltpu.sync_copy(den_sh.at[pl.ds(sid * ZB, ZB)],
                    den_hbm.at[cid, pl.ds(sid * ZB, ZB)])


def _sc_call(h, ei, ab_flat):
    mesh = plsc.VectorSubcoreMesh(core_axis_name="c", subcore_axis_name="s",
                                  num_cores=NC, num_subcores=NS)
    fn = pl.kernel(
        _sc_body,
        out_type=(
            jax.ShapeDtypeStruct((NC, N, D), jnp.float32),
            jax.ShapeDtypeStruct((NC, TBL), jnp.float32),
        ),
        mesh=mesh,
        compiler_params=pltpu.CompilerParams(needs_layout_passes=False,
                                             use_tc_tiling_on_sc=False),
        scratch_types=(
            pltpu.VMEM((C,), jnp.int32),
            pltpu.VMEM((C,), jnp.int32),
            pltpu.VMEM((C,), jnp.int32),
            pltpu.VMEM((C,), jnp.int32),
            pltpu.VMEM((2 * TBL,), jnp.float32),
            pltpu.VMEM((ZB,), jnp.float32),
            pltpu.VMEM((C, D), jnp.float32),
            pltpu.VMEM((C, D), jnp.float32),
            pltpu.VMEM((C,), jnp.int32),
            pltpu.VMEM((C,), jnp.int32),
            pltpu.VMEM((C + L,), jnp.float32),
            pltpu.VMEM((C + L,), jnp.float32),
            pltpu.VMEM_SHARED((N, D), jnp.float32),
            pltpu.VMEM_SHARED((TBL,), jnp.float32),
            pltpu.SemaphoreType.DMA,
            pltpu.SemaphoreType.DMA,
            pltpu.SemaphoreType.DMA,
            pltpu.SemaphoreType.DMA,
            pltpu.SemaphoreType.DMA,
            pltpu.SemaphoreType.DMA,
            pltpu.SemaphoreType.DMA,
            pltpu.SemaphoreType.DMA,
        ),
    )
    return fn(h, ei, ab_flat)


def _fin_body(p0_ref, p1_ref, d_ref, o_ref):
    s = p0_ref[0] + p1_ref[0]
    den = jnp.sum(d_ref[...], axis=1)
    o_ref[...] = s / jnp.maximum(den, 1e-9)[:, None]


def _fin_call(partials, denoms_t):
    blk = 400
    return pl.pallas_call(
        _fin_body,
        grid=(N // blk,),
        in_specs=[
            pl.BlockSpec((1, blk, D), lambda i: (0, i, 0)),
            pl.BlockSpec((1, blk, D), lambda i: (1, i, 0)),
            pl.BlockSpec((blk, NC), lambda i: (i, 0)),
        ],
        out_specs=pl.BlockSpec((blk, D), lambda i: (i, 0)),
        out_shape=jax.ShapeDtypeStruct((N, D), jnp.float32),
    )(partials, partials, denoms_t)


@jax.jit
def kernel(h, edge_index, W_att):
    ab = _ab_call(h, W_att.reshape(2, D))
    partials, denoms = _sc_call(h, edge_index, ab.reshape(2 * N))
    den_t = denoms.T[:N]
    return _fin_call(partials, den_t)


# speedup vs baseline: 2.2874x; 1.0001x over previous
import jax
import jax.numpy as jnp
from jax import lax
from jax.experimental import pallas as pl
from jax.experimental.pallas import tpu as pltpu
from jax.experimental.pallas import tpu_sc as plsc

N = 10000
E = 320000
D = 128
NEG_SLOPE = 0.2

NC = 2
NS = 16
L = 16
NW = NC * NS
EW = E // NW
C = 80
NCHUNK = EW // C
NPAIR = NCHUNK // 2
RPT8 = 624
TBL = 10240
ZB = TBL // NS


def _ab_body(h_ref, w_ref, o_ref):
    o_ref[...] = lax.dot_general(
        h_ref[...], w_ref[...], (((1,), (1,)), ((), ())),
        preferred_element_type=jnp.float32)


def _ab_call(h, w2):
    blk = 1000
    return pl.pallas_call(
        _ab_body,
        grid=(N // blk,),
        in_specs=[
            pl.BlockSpec((blk, D), lambda i: (i, 0)),
            pl.BlockSpec((2, D), lambda i: (0, 0)),
        ],
        out_specs=pl.BlockSpec((blk, 2), lambda i: (i, 0)),
        out_shape=jax.ShapeDtypeStruct((N, 2), jnp.float32),
    )(h, w2)


def _sc_body(h_hbm, ei_hbm, ab_hbm, part_hbm, den_hbm,
             src0, dst0, src1, dst1, ab_v, zbuf, rows0, rows1,
             scidx0, scidx1, pden0, pden1, acc, den_sh,
             gsem0, gsem1, isem0, isem1, ssem0, ssem1, dsem0, dsem1):
    cid = lax.axis_index("c")
    sid = lax.axis_index("s")
    wid = sid * NC + cid
    ebase = wid * EW

    pltpu.sync_copy(ab_hbm, ab_v.at[pl.ds(0, 2 * N)])

    def _zrow(i, carry):
        for j in range(D // L):
            rows0[i, pl.ds(j * L, L)] = jnp.zeros((L,), jnp.float32)
        return carry
    lax.fori_loop(0, C, _zrow, 0)

    def _zzb(i, carry):
        zbuf[pl.ds(i * L, L)] = jnp.zeros((L,), jnp.float32)
        return carry
    lax.fori_loop(0, ZB // L, _zzb, 0)

    base = sid * RPT8
    zcps = [pltpu.async_copy(rows0, acc.at[pl.ds(base + k * C, C)], gsem0)
            for k in range(RPT8 // C)]
    zcps.append(pltpu.async_copy(rows0.at[pl.ds(0, RPT8 % C)],
                                 acc.at[pl.ds(base + (RPT8 // C) * C,
                                              RPT8 % C)], gsem0))
    zcps.append(pltpu.async_copy(zbuf, den_sh.at[pl.ds(sid * ZB, ZB)], gsem0))

    @pl.when(sid == NS - 1)
    def _tail_zero():
        pltpu.sync_copy(rows0.at[pl.ds(0, N - NS * RPT8)],
                        acc.at[pl.ds(NS * RPT8, N - NS * RPT8)])
    for cp in zcps:
        cp.wait()
    plsc.subcore_barrier()

    def _process(src_r, dst_r, rows_r, scidx, pden, ssem, dsem):
        for g in range(C // L):
            sv = src_r[pl.ds(g * L, L)]
            dv = dst_r[pl.ds(g * L, L)]
            e = (plsc.load_gather(ab_v, [lax.shift_left(sv, 1)])
                 + plsc.load_gather(ab_v,
                                    [jnp.bitwise_or(lax.shift_left(dv, 1),
                                                    1)]))
            e = jnp.where(e >= 0, e, NEG_SLOPE * e)
            pden[pl.ds(g * L, L)] = jnp.exp(e)
            scidx[pl.ds(g * L, L)] = dv
        pltpu.async_copy(pden.at[pl.ds(0, C)], den_sh.at[scidx], dsem,
                         add=True)

        def _scale(i8, carry2):
            i = 8 * i8
            pis = [pden[pl.ds(i + u, L)][0] for u in range(8)]
            for u in range(8):
                for j in range(D // L):
                    rows_r[i + u, pl.ds(j * L, L)] = (
                        rows_r[i + u, pl.ds(j * L, L)] * pis[u])
            return carry2
        lax.fori_loop(0, C // 8, _scale, 0)

        pltpu.async_copy(rows_r, acc.at[scidx], ssem, add=True)

    pltpu.async_copy(ei_hbm.at[0, pl.ds(ebase, C)], src0, isem0).wait()
    pltpu.async_copy(ei_hbm.at[1, pl.ds(ebase, C)], dst0, isem0).wait()
    pltpu.async_copy(ei_hbm.at[0, pl.ds(ebase + C, C)], src1, isem1)
    pltpu.async_copy(ei_hbm.at[1, pl.ds(ebase + C, C)], dst1, isem1)
    pltpu.async_copy(h_hbm.at[src0], rows0, gsem0)

    def _drain_rows(rows_r, ssem):
        pltpu.make_async_copy(h_hbm.at[pl.ds(0, C)], rows_r, ssem).wait()

    def _drain_p(pden, dsem):
        pltpu.make_async_copy(ab_hbm.at[pl.ds(0, C)], pden.at[pl.ds(0, C)],
                              dsem).wait()

    def _pair(k, carry):
        c0 = 2 * k

        @pl.when(k > 0)
        def _dr1():
            _drain_rows(rows1, ssem1)
            _drain_p(pden1, dsem1)

        pltpu.make_async_copy(ei_hbm.at[0, pl.ds(0, C)], src1, isem1).wait()
        pltpu.make_async_copy(ei_hbm.at[1, pl.ds(0, C)], dst1, isem1).wait()
        pltpu.async_copy(h_hbm.at[src1], rows1, gsem1)

        @pl.when(k > 0)
        def _dr0():
            _drain_p(pden0, dsem0)

        pltpu.make_async_copy(h_hbm.at[pl.ds(0, C)], rows0, gsem0).wait()
        _process(src0, dst0, rows0, scidx0, pden0, ssem0, dsem0)
        pltpu.async_copy(ei_hbm.at[0, pl.ds(ebase + (c0 + 2) * C, C)],
                         src0, isem0)
        pltpu.async_copy(ei_hbm.at[1, pl.ds(ebase + (c0 + 2) * C, C)],
                         dst0, isem0)
        pltpu.make_async_copy(h_hbm.at[pl.ds(0, C)], rows1, gsem1).wait()
        _process(src1, dst1, rows1, scidx1, pden1, ssem1, dsem1)

        @pl.when(k < NPAIR - 1)
        def _pf1():
            pltpu.async_copy(ei_hbm.at[0, pl.ds(ebase + (c0 + 3) * C, C)],
                             src1, isem1)
            pltpu.async_copy(ei_hbm.at[1, pl.ds(ebase + (c0 + 3) * C, C)],
                             dst1, isem1)
        pltpu.make_async_copy(ei_hbm.at[0, pl.ds(0, C)], src0, isem0).wait()
        pltpu.make_async_copy(ei_hbm.at[1, pl.ds(0, C)], dst0, isem0).wait()
        _drain_rows(rows0, ssem0)
        pltpu.async_copy(h_hbm.at[src0], rows0, gsem0)
        return carry
    lax.fori_loop(0, NPAIR, _pair, 0)

    pltpu.make_async_copy(h_hbm.at[pl.ds(0, C)], rows0, gsem0).wait()
    _drain_p(pden0, dsem0)
    _process(src0, dst0, rows0, scidx0, pden0, ssem0, dsem0)
    _drain_rows(rows0, ssem0)
    _drain_p(pden0, dsem0)
    _drain_rows(rows1, ssem1)
    _drain_p(pden1, dsem1)

    plsc.subcore_barrier()

    pltpu.sync_copy(acc.at[pl.ds(base, RPT8)],
                    part_hbm.at[cid, pl.ds(base, RPT8)])

    @pl.when(sid == NS - 1)
    def _tail_out():
        pltpu.sync_copy(acc.at[pl.ds(NS * RPT8, N - NS * RPT8)],
                        part_hbm.at[cid, pl.ds(NS * RPT8, N - NS * RPT8)])

    pltpu.sync_copy(den_sh.at[pl.ds(sid * ZB, ZB)],
                    den_hbm.at[cid, pl.ds(sid * ZB, ZB)])


def _sc_call(h, ei, ab_flat):
    mesh = plsc.VectorSubcoreMesh(core_axis_name="c", subcore_axis_name="s",
                                  num_cores=NC, num_subcores=NS)
    fn = pl.kernel(
        _sc_body,
        out_type=(
            jax.ShapeDtypeStruct((NC, N, D), jnp.float32),
            jax.ShapeDtypeStruct((NC, TBL), jnp.float32),
        ),
        mesh=mesh,
        compiler_params=pltpu.CompilerParams(needs_layout_passes=False,
                                             use_tc_tiling_on_sc=False),
        scratch_types=(
            pltpu.VMEM((C,), jnp.int32),
            pltpu.VMEM((C,), jnp.int32),
            pltpu.VMEM((C,), jnp.int32),
            pltpu.VMEM((C,), jnp.int32),
            pltpu.VMEM((2 * TBL,), jnp.float32),
            pltpu.VMEM((ZB,), jnp.float32),
            pltpu.VMEM((C, D), jnp.float32),
            pltpu.VMEM((C, D), jnp.float32),
            pltpu.VMEM((C,), jnp.int32),
            pltpu.VMEM((C,), jnp.int32),
            pltpu.VMEM((C + L,), jnp.float32),
            pltpu.VMEM((C + L,), jnp.float32),
            pltpu.VMEM_SHARED((N, D), jnp.float32),
            pltpu.VMEM_SHARED((TBL,), jnp.float32),
            pltpu.SemaphoreType.DMA,
            pltpu.SemaphoreType.DMA,
            pltpu.SemaphoreType.DMA,
            pltpu.SemaphoreType.DMA,
            pltpu.SemaphoreType.DMA,
            pltpu.SemaphoreType.DMA,
            pltpu.SemaphoreType.DMA,
            pltpu.SemaphoreType.DMA,
        ),
    )
    return fn(h, ei, ab_flat)


def _fin_body(p0_ref, p1_ref, d_ref, o_ref):
    s = p0_ref[0] + p1_ref[0]
    den = jnp.sum(d_ref[...], axis=1)
    o_ref[...] = s / jnp.maximum(den, 1e-9)[:, None]


def _fin_call(partials, denoms_t):
    blk = 400
    return pl.pallas_call(
        _fin_body,
        grid=(N // blk,),
        in_specs=[
            pl.BlockSpec((1, blk, D), lambda i: (0, i, 0)),
            pl.BlockSpec((1, blk, D), lambda i: (1, i, 0)),
            pl.BlockSpec((blk, NC), lambda i: (i, 0)),
        ],
        out_specs=pl.BlockSpec((blk, D), lambda i: (i, 0)),
        out_shape=jax.ShapeDtypeStruct((N, D), jnp.float32),
    )(partials, partials, denoms_t)


@jax.jit
def kernel(h, edge_index, W_att):
    ab = _ab_call(h, W_att.reshape(2, D))
    partials, denoms = _sc_call(h, edge_index, ab.reshape(2 * N))
    den_t = denoms.T[:N]
    return _fin_call(partials, den_t)
